# Initial kernel scaffold; baseline (speedup 1.0000x reference)
import jax
import jax.numpy as jnp
from jax.experimental import pallas as pl


def kernel(x, edge_index, w_in, b_in, w_t1, b_t1, w_t2, b_t2, wg0, as0, ad0, bg0, bn_g0, bn_b0, bn_m0, bn_v0, wg1, as1, ad1, bg1, bn_g1, bn_b1, bn_m1, bn_v1, wg2, as2, ad2, bg2, bn_g2, bn_b2, bn_m2, bn_v2, w_c1, b_c1, w_c2, b_c2):
    raise NotImplementedError("write your pallas kernel here")


# speedup vs baseline: 19.8109x; 19.8109x over previous
import functools

import jax
import jax.numpy as jnp
from jax import lax
from jax.experimental import pallas as pl
from jax.experimental.pallas import tpu as pltpu
from jax.experimental.pallas import tpu_sc as plsc

N = 10000
D_IN = 128
HID = 128
HEADS = 8
E = 160000
EF = E + N
NP = 10240
EP = 172032
NC, NS = 2, 16
NW = NC * NS
EPW = EP // NW
STRIPE = NP // NS

_R = 1024
_G = NP // _R


def _blk(shape):
    return pl.BlockSpec(shape, lambda i: (0,) * len(shape))


def _rows(cols):
    return pl.BlockSpec((_R, cols), lambda i: (i, 0))


def _dot(a, b):
    return jnp.dot(a, b, preferred_element_type=jnp.float32)


def _tc0_body(x_ref, w_in_ref, b_in_ref, w_t1_ref, b_t1_ref, w_t2_ref,
              b_t2_ref, wg_ref, asm_ref, adm_ref,
              h_ref, ht_ref, xw_ref, als_ref, ald_ref):
    h = _dot(x_ref[...], w_in_ref[...]) + b_in_ref[...]
    h_ref[...] = h
    t = jnp.maximum(_dot(h, w_t1_ref[...]) + b_t1_ref[...], 0.0)
    ht_ref[...] = jnp.maximum(_dot(t, w_t2_ref[...]) + b_t2_ref[...], 0.0)
    xw = _dot(h, wg_ref[...])
    xw_ref[...] = xw
    als_ref[...] = _dot(xw, asm_ref[...])
    ald_ref[...] = _dot(xw, adm_ref[...])


def _tc0(x_p, w_in, b_in, w_t1, b_t1, w_t2, b_t2, wg0, asm0, adm0):
    return pl.pallas_call(
        _tc0_body,
        grid=(_G,),
        in_specs=[_rows(HID), _blk((HID, HID)), _blk((1, HID)),
                  _blk((HID, HID)), _blk((1, HID)), _blk((HID, HID)),
                  _blk((1, HID)), _blk((HID, HID)), _blk((HID, 16)),
                  _blk((HID, 16))],
        out_specs=[_rows(HID), _rows(HID), _rows(HID), _rows(16), _rows(16)],
        out_shape=[jax.ShapeDtypeStruct((NP, HID), jnp.float32)] * 3
        + [jax.ShapeDtypeStruct((NP, 16), jnp.float32)] * 2,
    )(x_p, w_in, b_in, w_t1, b_t1, w_t2, b_t2, wg0, asm0, adm0)


def _tc_mid_body(oa_ref, ob_ref, bg_ref, g_ref, b_ref, m_ref, v_ref,
                 wg_ref, asm_ref, adm_ref, xw_ref, als_ref, ald_ref):
    gs = oa_ref[...] + ob_ref[...] + bg_ref[...]
    xh = (gs - m_ref[...]) * lax.rsqrt(v_ref[...] + 1e-5) * g_ref[...] + b_ref[...]
    xh = jnp.where(xh > 0.0, xh, jnp.exp(xh) - 1.0)
    xw = _dot(xh, wg_ref[...])
    xw_ref[...] = xw
    als_ref[...] = _dot(xw, asm_ref[...])
    ald_ref[...] = _dot(xw, adm_ref[...])


def _tc_mid(oa, ob, bg, g, b, m, v, wg, asm, adm):
    w = wg.shape[1]
    return pl.pallas_call(
        _tc_mid_body,
        grid=(_G,),
        in_specs=[_rows(HID), _rows(HID)] + [_blk((1, HID))] * 5
        + [_blk((HID, w)), _blk((w, 16)), _blk((w, 16))],
        out_specs=[_rows(w), _rows(16), _rows(16)],
        out_shape=[jax.ShapeDtypeStruct((NP, w), jnp.float32),
                   jax.ShapeDtypeStruct((NP, 16), jnp.float32),
                   jax.ShapeDtypeStruct((NP, 16), jnp.float32)],
    )(oa, ob, bg, g, b, m, v, wg, asm, adm)


def _tc3_body(oa_ref, ob_ref, bg_ref, g_ref, b_ref, m_ref, v_ref, ht_ref,
              wc1a_ref, wc1b_ref, bc1_ref, wc2_ref, bc2_ref, out_ref):
    gs = oa_ref[...] + ob_ref[...] + bg_ref[...]
    hg = (gs - m_ref[...]) * lax.rsqrt(v_ref[...] + 1e-5) * g_ref[...] + b_ref[...]
    z = jnp.maximum(_dot(hg, wc1a_ref[...]) + _dot(ht_ref[...], wc1b_ref[...])
                    + bc1_ref[...], 0.0)
    out_ref[...] = _dot(z, wc2_ref[...]) + bc2_ref[...]


def _tc3(oa, ob, bg, g, b, m, v, ht, wc1a, wc1b, bc1, wc2, bc2):
    return pl.pallas_call(
        _tc3_body,
        grid=(_G,),
        in_specs=[_rows(HID), _rows(HID)] + [_blk((1, HID))] * 5
        + [_rows(HID), _blk((HID, HID)), _blk((HID, HID)), _blk((1, HID)),
           _blk((HID, 2)), _blk((1, 2))],
        out_specs=[_rows(2)],
        out_shape=[jax.ShapeDtypeStruct((NP, 2), jnp.float32)],
    )(oa, ob, bg, g, b, m, v, ht, wc1a, wc1b, bc1, wc2, bc2)


_MESH = plsc.VectorSubcoreMesh(core_axis_name="c", subcore_axis_name="s")

_CHA = 128
_NCHA = EPW // _CHA


@functools.partial(
    pl.kernel,
    out_type=(
        jax.ShapeDtypeStruct((EP, 16), jnp.float32),
        jax.ShapeDtypeStruct((NP, 16), jnp.float32),
        jax.ShapeDtypeStruct((NP, 16), jnp.float32),
    ),
    mesh=_MESH,
    compiler_params=pltpu.CompilerParams(use_tc_tiling_on_sc=False),
    scratch_types=[
        pltpu.VMEM((_CHA,), jnp.int32),
        pltpu.VMEM((_CHA,), jnp.int32),
        pltpu.VMEM((_CHA, 16), jnp.float32),
        pltpu.VMEM((_CHA, 16), jnp.float32),
        pltpu.VMEM((_CHA, 16), jnp.float32),
        pltpu.VMEM_SHARED((NP, 16), jnp.float32),
        pltpu.SemaphoreType.DMA,
    ],
)
def _sc_phase_a(als_hbm, ald_hbm, src_hbm, dst_hbm, zero16_hbm,
                ee_hbm, den0_hbm, den1_hbm,
                src_v, dst_v, gs_v, gd_v, ee_v, den_acc, sem):
    c = lax.axis_index("c")
    s = lax.axis_index("s")
    wid = s * NC + c
    stripe = pl.ds(s * STRIPE, STRIPE)
    pltpu.sync_copy(zero16_hbm.at[stripe], den_acc.at[stripe])
    plsc.subcore_barrier()

    def chunk_body(k, _):
        base = wid * EPW + k * _CHA
        pltpu.sync_copy(src_hbm.at[pl.ds(base, _CHA)], src_v)
        pltpu.sync_copy(dst_hbm.at[pl.ds(base, _CHA)], dst_v)
        pltpu.async_copy(als_hbm.at[src_v], gs_v, sem).wait()
        pltpu.async_copy(ald_hbm.at[dst_v], gd_v, sem).wait()

        def row_body(r, _):
            e = gs_v[r, :] + gd_v[r, :]
            e = jnp.where(e >= 0.0, e, 0.2 * e)
            ee_v[r, :] = jnp.exp(e)
            return 0

        lax.fori_loop(0, _CHA, row_body, 0, unroll=4)
        pltpu.sync_copy(ee_v, ee_hbm.at[pl.ds(base, _CHA)])
        pltpu.sync_copy(ee_v, den_acc.at[dst_v], add=True)
        return 0

    lax.fori_loop(0, _NCHA, chunk_body, 0)
    plsc.subcore_barrier()

    @pl.when(c == 0)
    def _():
        pltpu.sync_copy(den_acc.at[stripe], den0_hbm.at[stripe])

    @pl.when(c == 1)
    def _():
        pltpu.sync_copy(den_acc.at[stripe], den1_hbm.at[stripe])


def _mk_phase_b(od):
    w = HEADS * od
    ch = 128 if od <= 16 else 32
    nch = EPW // ch
    scale = 1.0 if od <= 16 else 1.0 / HEADS

    @functools.partial(
        pl.kernel,
        out_type=(
            jax.ShapeDtypeStruct((NP, HID), jnp.float32),
            jax.ShapeDtypeStruct((NP, HID), jnp.float32),
        ),
        mesh=_MESH,
        compiler_params=pltpu.CompilerParams(use_tc_tiling_on_sc=False),
        scratch_types=[
            pltpu.VMEM((ch,), jnp.int32),
            pltpu.VMEM((ch,), jnp.int32),
            pltpu.VMEM((ch, w), jnp.float32),
            pltpu.VMEM((ch, 16), jnp.float32),
            pltpu.VMEM((ch, 16), jnp.float32),
            pltpu.VMEM((ch, 16), jnp.float32),
            pltpu.VMEM((ch, HID), jnp.float32),
            pltpu.VMEM_SHARED((NP, HID), jnp.float32),
            pltpu.SemaphoreType.DMA,
        ],
    )
    def phase_b(xw_hbm, ee_hbm, den0_hbm, den1_hbm, src_hbm, dst_hbm,
                zero128_hbm, out0_hbm, out1_hbm,
                src_v, dst_v, xw_v, ee_v, da_v, db_v, ct_v, out_acc,
                sem):
        c = lax.axis_index("c")
        s = lax.axis_index("s")
        wid = s * NC + c
        stripe = pl.ds(s * STRIPE, STRIPE)
        pltpu.sync_copy(zero128_hbm.at[stripe], out_acc.at[stripe])
        plsc.subcore_barrier()

        def chunk_body(k, _):
            base = wid * EPW + k * ch
            pltpu.sync_copy(src_hbm.at[pl.ds(base, ch)], src_v)
            pltpu.sync_copy(dst_hbm.at[pl.ds(base, ch)], dst_v)
            pltpu.async_copy(xw_hbm.at[src_v], xw_v, sem).wait()
            pltpu.async_copy(den0_hbm.at[dst_v], da_v, sem).wait()
            pltpu.async_copy(den1_hbm.at[dst_v], db_v, sem).wait()
            pltpu.sync_copy(ee_hbm.at[pl.ds(base, ch)], ee_v)

            def row_body(r, _):
                cf = ee_v[r, :] / (da_v[r, :] + db_v[r, :] + 1e-16) * scale
                if od <= 16:
                    for j in range(HEADS):
                        ct_v[r, pl.ds(16 * j, 16)] = (
                            cf[j] * xw_v[r, pl.ds(16 * j, 16)])
                else:
                    for j in range(8):
                        acc = cf[0] * xw_v[r, pl.ds(16 * j, 16)]
                        for h in range(1, HEADS):
                            acc = acc + cf[h] * xw_v[
                                r, pl.ds(h * HID + 16 * j, 16)]
                        ct_v[r, pl.ds(16 * j, 16)] = acc
                return 0

            lax.fori_loop(0, ch, row_body, 0)
            pltpu.sync_copy(ct_v, out_acc.at[dst_v], add=True)
            return 0

        lax.fori_loop(0, nch, chunk_body, 0)
        plsc.subcore_barrier()

        @pl.when(c == 0)
        def _():
            pltpu.sync_copy(out_acc.at[stripe], out0_hbm.at[stripe])

        @pl.when(c == 1)
        def _():
            pltpu.sync_copy(out_acc.at[stripe], out1_hbm.at[stripe])

    return phase_b


_sc_phase_b16 = _mk_phase_b(16)
_sc_phase_b128 = _mk_phase_b(128)


def _att_mat(a):
    od = a.shape[1]
    mask = jnp.kron(jnp.eye(HEADS, dtype=jnp.float32), jnp.ones((od, 1), jnp.float32))
    vals = jnp.tile(a.T, (HEADS, 1))
    m = mask * vals
    return jnp.concatenate([m, m], axis=1)


def kernel(x, edge_index, w_in, b_in, w_t1, b_t1, w_t2, b_t2, wg0, as0, ad0,
           bg0, bn_g0, bn_b0, bn_m0, bn_v0, wg1, as1, ad1, bg1, bn_g1, bn_b1,
           bn_m1, bn_v1, wg2, as2, ad2, bg2, bn_g2, bn_b2, bn_m2, bn_v2,
           w_c1, b_c1, w_c2, b_c2):
    f32 = jnp.float32
    x_p = jnp.pad(x, ((0, NP - N), (0, 0)))
    loops = jnp.arange(N, dtype=jnp.int32)
    padi = jnp.full((EP - EF,), N, jnp.int32)
    src = jnp.concatenate([edge_index[0].astype(jnp.int32), loops, padi])
    dst = jnp.concatenate([edge_index[1].astype(jnp.int32), loops, padi])

    row = lambda v: v.reshape(1, -1).astype(f32)
    z16 = jnp.zeros((NP, 16), f32)
    z128 = jnp.zeros((NP, HID), f32)

    h, ht, xw0, als0, ald0 = _tc0(
        x_p, w_in, row(b_in), w_t1, row(b_t1), w_t2, row(b_t2), wg0,
        _att_mat(as0), _att_mat(ad0))

    ee0, dena0, denb0 = _sc_phase_a(als0, ald0, src, dst, z16)
    outa0, outb0 = _sc_phase_b16(xw0, ee0, dena0, denb0, src, dst, z128)

    xw1, als1, ald1 = _tc_mid(
        outa0, outb0, row(bg0), row(bn_g0), row(bn_b0), row(bn_m0),
        row(bn_v0), wg1, _att_mat(as1), _att_mat(ad1))

    ee1, dena1, denb1 = _sc_phase_a(als1, ald1, src, dst, z16)
    outa1, outb1 = _sc_phase_b16(xw1, ee1, dena1, denb1, src, dst, z128)

    xw2, als2, ald2 = _tc_mid(
        outa1, outb1, row(bg1), row(bn_g1), row(bn_b1), row(bn_m1),
        row(bn_v1), wg2, _att_mat(as2), _att_mat(ad2))

    ee2, dena2, denb2 = _sc_phase_a(als2, ald2, src, dst, z16)
    outa2, outb2 = _sc_phase_b128(xw2, ee2, dena2, denb2, src, dst, z128)

    out = _tc3(outa2, outb2, row(bg2), row(bn_g2), row(bn_b2), row(bn_m2),
               row(bn_v2), ht, w_c1[:HID], w_c1[HID:], row(b_c1), w_c2,
               row(b_c2))[0]
    return out[:N]


# speedup vs baseline: 29.2159x; 1.4747x over previous
import functools

import jax
import jax.numpy as jnp
from jax import lax
from jax.experimental import pallas as pl
from jax.experimental.pallas import tpu as pltpu
from jax.experimental.pallas import tpu_sc as plsc

N = 10000
D_IN = 128
HID = 128
HEADS = 8
E = 160000
EF = E + N
NP = 10240
EP = 172032
NC, NS = 2, 16
NW = NC * NS
EPW = EP // NW
STRIPE = NP // NS

_R = 1024
_G = NP // _R


def _blk(shape):
    return pl.BlockSpec(shape, lambda i: (0,) * len(shape))


def _rows(cols):
    return pl.BlockSpec((_R, cols), lambda i: (i, 0))


def _dot(a, b):
    return jnp.dot(a, b, preferred_element_type=jnp.float32)


def _tc0_body(x_ref, w_in_ref, b_in_ref, w_t1_ref, b_t1_ref, w_t2_ref,
              b_t2_ref, wg_ref, asm_ref, adm_ref,
              h_ref, ht_ref, xw_ref, als_ref, ald_ref):
    h = _dot(x_ref[...], w_in_ref[...]) + b_in_ref[...]
    h_ref[...] = h
    t = jnp.maximum(_dot(h, w_t1_ref[...]) + b_t1_ref[...], 0.0)
    ht_ref[...] = jnp.maximum(_dot(t, w_t2_ref[...]) + b_t2_ref[...], 0.0)
    xw = _dot(h, wg_ref[...])
    xw_ref[...] = xw
    als_ref[...] = _dot(xw, asm_ref[...])
    ald_ref[...] = _dot(xw, adm_ref[...])


def _tc0(x_p, w_in, b_in, w_t1, b_t1, w_t2, b_t2, wg0, asm0, adm0):
    return pl.pallas_call(
        _tc0_body,
        grid=(_G,),
        in_specs=[_rows(HID), _blk((HID, HID)), _blk((1, HID)),
                  _blk((HID, HID)), _blk((1, HID)), _blk((HID, HID)),
                  _blk((1, HID)), _blk((HID, HID)), _blk((HID, 16)),
                  _blk((HID, 16))],
        out_specs=[_rows(HID), _rows(HID), _rows(HID), _rows(16), _rows(16)],
        out_shape=[jax.ShapeDtypeStruct((NP, HID), jnp.float32)] * 3
        + [jax.ShapeDtypeStruct((NP, 16), jnp.float32)] * 2,
    )(x_p, w_in, b_in, w_t1, b_t1, w_t2, b_t2, wg0, asm0, adm0)


def _tc_mid_body(oa_ref, ob_ref, bg_ref, g_ref, b_ref, m_ref, v_ref,
                 wg_ref, asm_ref, adm_ref, xw_ref, als_ref, ald_ref):
    gs = oa_ref[...] + ob_ref[...] + bg_ref[...]
    xh = (gs - m_ref[...]) * lax.rsqrt(v_ref[...] + 1e-5) * g_ref[...] + b_ref[...]
    xh = jnp.where(xh > 0.0, xh, jnp.exp(xh) - 1.0)
    xw = _dot(xh, wg_ref[...])
    xw_ref[...] = xw
    als_ref[...] = _dot(xw, asm_ref[...])
    ald_ref[...] = _dot(xw, adm_ref[...])


def _tc_mid(oa, ob, bg, g, b, m, v, wg, asm, adm):
    w = wg.shape[1]
    return pl.pallas_call(
        _tc_mid_body,
        grid=(_G,),
        in_specs=[_rows(HID), _rows(HID)] + [_blk((1, HID))] * 5
        + [_blk((HID, w)), _blk((w, 16)), _blk((w, 16))],
        out_specs=[_rows(w), _rows(16), _rows(16)],
        out_shape=[jax.ShapeDtypeStruct((NP, w), jnp.float32),
                   jax.ShapeDtypeStruct((NP, 16), jnp.float32),
                   jax.ShapeDtypeStruct((NP, 16), jnp.float32)],
    )(oa, ob, bg, g, b, m, v, wg, asm, adm)


def _tc3_body(oa_ref, ob_ref, bg_ref, g_ref, b_ref, m_ref, v_ref, ht_ref,
              wc1a_ref, wc1b_ref, bc1_ref, wc2_ref, bc2_ref, out_ref):
    gs = oa_ref[...] + ob_ref[...] + bg_ref[...]
    hg = (gs - m_ref[...]) * lax.rsqrt(v_ref[...] + 1e-5) * g_ref[...] + b_ref[...]
    z = jnp.maximum(_dot(hg, wc1a_ref[...]) + _dot(ht_ref[...], wc1b_ref[...])
                    + bc1_ref[...], 0.0)
    out_ref[...] = _dot(z, wc2_ref[...]) + bc2_ref[...]


def _tc3(oa, ob, bg, g, b, m, v, ht, wc1a, wc1b, bc1, wc2, bc2):
    return pl.pallas_call(
        _tc3_body,
        grid=(_G,),
        in_specs=[_rows(HID), _rows(HID)] + [_blk((1, HID))] * 5
        + [_rows(HID), _blk((HID, HID)), _blk((HID, HID)), _blk((1, HID)),
           _blk((HID, 2)), _blk((1, 2))],
        out_specs=[_rows(2)],
        out_shape=[jax.ShapeDtypeStruct((NP, 2), jnp.float32)],
    )(oa, ob, bg, g, b, m, v, ht, wc1a, wc1b, bc1, wc2, bc2)


_MESH = plsc.VectorSubcoreMesh(core_axis_name="c", subcore_axis_name="s")

_CHA = 128
_NCHA = EPW // _CHA


@functools.partial(
    pl.kernel,
    out_type=(
        jax.ShapeDtypeStruct((EP, 16), jnp.float32),
        jax.ShapeDtypeStruct((NP, 16), jnp.float32),
        jax.ShapeDtypeStruct((NP, 16), jnp.float32),
    ),
    mesh=_MESH,
    compiler_params=pltpu.CompilerParams(use_tc_tiling_on_sc=False),
    scratch_types=[
        pltpu.VMEM((2, _CHA), jnp.int32),
        pltpu.VMEM((2, _CHA), jnp.int32),
        pltpu.VMEM((2, _CHA, 16), jnp.float32),
        pltpu.VMEM((2, _CHA, 16), jnp.float32),
        pltpu.VMEM((_CHA, 16), jnp.float32),
        pltpu.VMEM_SHARED((NP, 16), jnp.float32),
        pltpu.SemaphoreType.DMA,
        pltpu.SemaphoreType.DMA,
    ],
)
def _sc_phase_a(als_hbm, ald_hbm, src_hbm, dst_hbm, zero16_hbm,
                ee_hbm, den0_hbm, den1_hbm,
                src_v, dst_v, gs_v, gd_v, ee_v, den_acc, sem0, sem1):
    c = lax.axis_index("c")
    s = lax.axis_index("s")
    wid = s * NC + c
    stripe = pl.ds(s * STRIPE, STRIPE)
    pltpu.sync_copy(zero16_hbm.at[stripe], den_acc.at[stripe])
    plsc.subcore_barrier()
    sems = (sem0, sem1)

    def issue(k, b):
        base = wid * EPW + k * _CHA
        pltpu.sync_copy(src_hbm.at[pl.ds(base, _CHA)], src_v.at[b])
        pltpu.sync_copy(dst_hbm.at[pl.ds(base, _CHA)], dst_v.at[b])
        pltpu.async_copy(als_hbm.at[src_v.at[b]], gs_v.at[b], sems[b])
        pltpu.async_copy(ald_hbm.at[dst_v.at[b]], gd_v.at[b], sems[b])

    def wait(b):
        pltpu.make_async_copy(als_hbm.at[src_v.at[b]], gs_v.at[b], sems[b]).wait()
        pltpu.make_async_copy(ald_hbm.at[dst_v.at[b]], gd_v.at[b], sems[b]).wait()

    def work(k, b):
        def row_body(r, _):
            e = gs_v[b, r, :] + gd_v[b, r, :]
            e = jnp.where(e >= 0.0, e, 0.2 * e)
            ee_v[r, :] = jnp.exp(e)
            return 0

        lax.fori_loop(0, _CHA, row_body, 0, unroll=4)
        base = wid * EPW + k * _CHA
        pltpu.sync_copy(ee_v, ee_hbm.at[pl.ds(base, _CHA)])
        pltpu.sync_copy(ee_v, den_acc.at[dst_v.at[b]], add=True)

    issue(0, 0)

    def pair_body(p, _):
        issue(2 * p + 1, 1)
        wait(0)
        work(2 * p, 0)

        @pl.when(2 * p + 2 < _NCHA)
        def _():
            issue(2 * p + 2, 0)

        wait(1)
        work(2 * p + 1, 1)
        return 0

    lax.fori_loop(0, _NCHA // 2, pair_body, 0)
    plsc.subcore_barrier()

    @pl.when(c == 0)
    def _():
        pltpu.sync_copy(den_acc.at[stripe], den0_hbm.at[stripe])

    @pl.when(c == 1)
    def _():
        pltpu.sync_copy(den_acc.at[stripe], den1_hbm.at[stripe])


def _mk_phase_b(od):
    w = HEADS * od
    ch = 64 if od <= 16 else 16
    nch = EPW // ch
    scale = 1.0 if od <= 16 else 1.0 / HEADS

    @functools.partial(
        pl.kernel,
        out_type=(
            jax.ShapeDtypeStruct((NP, HID), jnp.float32),
            jax.ShapeDtypeStruct((NP, HID), jnp.float32),
        ),
        mesh=_MESH,
        compiler_params=pltpu.CompilerParams(use_tc_tiling_on_sc=False),
        scratch_types=[
            pltpu.VMEM((2, ch), jnp.int32),
            pltpu.VMEM((2, ch), jnp.int32),
            pltpu.VMEM((2, ch, w), jnp.float32),
            pltpu.VMEM((2, ch, 16), jnp.float32),
            pltpu.VMEM((2, ch, 16), jnp.float32),
            pltpu.VMEM((2, ch, 16), jnp.float32),
            pltpu.VMEM((ch, HID), jnp.float32),
            pltpu.VMEM_SHARED((NP, HID), jnp.float32),
            pltpu.SemaphoreType.DMA,
            pltpu.SemaphoreType.DMA,
        ],
    )
    def phase_b(xw_hbm, ee_hbm, den0_hbm, den1_hbm, src_hbm, dst_hbm,
                zero128_hbm, out0_hbm, out1_hbm,
                src_v, dst_v, xw_v, ee_v, da_v, db_v, ct_v, out_acc,
                sem0, sem1):
        c = lax.axis_index("c")
        s = lax.axis_index("s")
        wid = s * NC + c
        stripe = pl.ds(s * STRIPE, STRIPE)
        pltpu.sync_copy(zero128_hbm.at[stripe], out_acc.at[stripe])
        plsc.subcore_barrier()
        sems = (sem0, sem1)

        def issue(k, b):
            base = wid * EPW + k * ch
            pltpu.sync_copy(src_hbm.at[pl.ds(base, ch)], src_v.at[b])
            pltpu.sync_copy(dst_hbm.at[pl.ds(base, ch)], dst_v.at[b])
            pltpu.async_copy(xw_hbm.at[src_v.at[b]], xw_v.at[b], sems[b])
            pltpu.async_copy(den0_hbm.at[dst_v.at[b]], da_v.at[b], sems[b])
            pltpu.async_copy(den1_hbm.at[dst_v.at[b]], db_v.at[b], sems[b])
            pltpu.async_copy(ee_hbm.at[pl.ds(base, ch)], ee_v.at[b], sems[b])

        def wait(k, b):
            base = wid * EPW + k * ch
            pltpu.make_async_copy(xw_hbm.at[src_v.at[b]], xw_v.at[b],
                                  sems[b]).wait()
            pltpu.make_async_copy(den0_hbm.at[dst_v.at[b]], da_v.at[b],
                                  sems[b]).wait()
            pltpu.make_async_copy(den1_hbm.at[dst_v.at[b]], db_v.at[b],
                                  sems[b]).wait()
            pltpu.make_async_copy(ee_hbm.at[pl.ds(base, ch)], ee_v.at[b],
                                  sems[b]).wait()

        def work(b):
            def row_body(r, _):
                cf = (ee_v[b, r, :]
                      / (da_v[b, r, :] + db_v[b, r, :] + 1e-16) * scale)
                if od <= 16:
                    for j in range(HEADS):
                        ct_v[r, pl.ds(16 * j, 16)] = (
                            cf[j] * xw_v[b, r, pl.ds(16 * j, 16)])
                else:
                    for j in range(8):
                        acc = cf[0] * xw_v[b, r, pl.ds(16 * j, 16)]
                        for h in range(1, HEADS):
                            acc = acc + cf[h] * xw_v[
                                b, r, pl.ds(h * HID + 16 * j, 16)]
                        ct_v[r, pl.ds(16 * j, 16)] = acc
                return 0

            lax.fori_loop(0, ch, row_body, 0)
            pltpu.sync_copy(ct_v, out_acc.at[dst_v.at[b]], add=True)

        issue(0, 0)

        def pair_body(p, _):
            issue(2 * p + 1, 1)
            wait(2 * p, 0)
            work(0)

            @pl.when(2 * p + 2 < nch)
            def _():
                issue(2 * p + 2, 0)

            wait(2 * p + 1, 1)
            work(1)
            return 0

        lax.fori_loop(0, nch // 2, pair_body, 0)
        plsc.subcore_barrier()

        @pl.when(c == 0)
        def _():
            pltpu.sync_copy(out_acc.at[stripe], out0_hbm.at[stripe])

        @pl.when(c == 1)
        def _():
            pltpu.sync_copy(out_acc.at[stripe], out1_hbm.at[stripe])

    return phase_b


_sc_phase_b16 = _mk_phase_b(16)
_sc_phase_b128 = _mk_phase_b(128)


def _att_mat(a):
    od = a.shape[1]
    mask = jnp.kron(jnp.eye(HEADS, dtype=jnp.float32), jnp.ones((od, 1), jnp.float32))
    vals = jnp.tile(a.T, (HEADS, 1))
    m = mask * vals
    return jnp.concatenate([m, m], axis=1)


def kernel(x, edge_index, w_in, b_in, w_t1, b_t1, w_t2, b_t2, wg0, as0, ad0,
           bg0, bn_g0, bn_b0, bn_m0, bn_v0, wg1, as1, ad1, bg1, bn_g1, bn_b1,
           bn_m1, bn_v1, wg2, as2, ad2, bg2, bn_g2, bn_b2, bn_m2, bn_v2,
           w_c1, b_c1, w_c2, b_c2):
    f32 = jnp.float32
    x_p = jnp.pad(x, ((0, NP - N), (0, 0)))
    loops = jnp.arange(N, dtype=jnp.int32)
    padi = jnp.full((EP - EF,), N, jnp.int32)
    src = jnp.concatenate([edge_index[0].astype(jnp.int32), loops, padi])
    dst = jnp.concatenate([edge_index[1].astype(jnp.int32), loops, padi])

    row = lambda v: v.reshape(1, -1).astype(f32)
    z16 = jnp.zeros((NP, 16), f32)
    z128 = jnp.zeros((NP, HID), f32)

    h, ht, xw0, als0, ald0 = _tc0(
        x_p, w_in, row(b_in), w_t1, row(b_t1), w_t2, row(b_t2), wg0,
        _att_mat(as0), _att_mat(ad0))

    ee0, dena0, denb0 = _sc_phase_a(als0, ald0, src, dst, z16)
    outa0, outb0 = _sc_phase_b16(xw0, ee0, dena0, denb0, src, dst, z128)

    xw1, als1, ald1 = _tc_mid(
        outa0, outb0, row(bg0), row(bn_g0), row(bn_b0), row(bn_m0),
        row(bn_v0), wg1, _att_mat(as1), _att_mat(ad1))

    ee1, dena1, denb1 = _sc_phase_a(als1, ald1, src, dst, z16)
    outa1, outb1 = _sc_phase_b16(xw1, ee1, dena1, denb1, src, dst, z128)

    xw2, als2, ald2 = _tc_mid(
        outa1, outb1, row(bg1), row(bn_g1), row(bn_b1), row(bn_m1),
        row(bn_v1), wg2, _att_mat(as2), _att_mat(ad2))

    ee2, dena2, denb2 = _sc_phase_a(als2, ald2, src, dst, z16)
    outa2, outb2 = _sc_phase_b128(xw2, ee2, dena2, denb2, src, dst, z128)

    out = _tc3(outa2, outb2, row(bg2), row(bn_g2), row(bn_b2), row(bn_m2),
               row(bn_v2), ht, w_c1[:HID], w_c1[HID:], row(b_c1), w_c2,
               row(b_c2))[0]
    return out[:N]


# speedup vs baseline: 38.5734x; 1.3203x over previous
import functools

import jax
import jax.numpy as jnp
from jax import lax
from jax.experimental import pallas as pl
from jax.experimental.pallas import tpu as pltpu
from jax.experimental.pallas import tpu_sc as plsc

N = 10000
D_IN = 128
HID = 128
HEADS = 8
E = 160000
EF = E + N
NP = 10240
EP = 172032
NC, NS = 2, 16
NW = NC * NS
EPW = EP // NW
STRIPE = NP // NS

_R = 1024
_G = NP // _R


def _blk(shape):
    return pl.BlockSpec(shape, lambda i: (0,) * len(shape))


def _rows(cols):
    return pl.BlockSpec((_R, cols), lambda i: (i, 0))


def _dot(a, b):
    return jnp.dot(a, b, preferred_element_type=jnp.float32)


def _tc0_body(x_ref, w_in_ref, b_in_ref, w_t1_ref, b_t1_ref, w_t2_ref,
              b_t2_ref, wg_ref, asm_ref, adm_ref,
              h_ref, ht_ref, xw_ref, als_ref, ald_ref):
    h = _dot(x_ref[...], w_in_ref[...]) + b_in_ref[...]
    h_ref[...] = h
    t = jnp.maximum(_dot(h, w_t1_ref[...]) + b_t1_ref[...], 0.0)
    ht_ref[...] = jnp.maximum(_dot(t, w_t2_ref[...]) + b_t2_ref[...], 0.0)
    xw = _dot(h, wg_ref[...])
    xw_ref[...] = xw
    als_ref[...] = _dot(xw, asm_ref[...])
    ald_ref[...] = _dot(xw, adm_ref[...])


def _tc0(x_p, w_in, b_in, w_t1, b_t1, w_t2, b_t2, wg0, asm0, adm0):
    return pl.pallas_call(
        _tc0_body,
        grid=(_G,),
        in_specs=[_rows(HID), _blk((HID, HID)), _blk((1, HID)),
                  _blk((HID, HID)), _blk((1, HID)), _blk((HID, HID)),
                  _blk((1, HID)), _blk((HID, HID)), _blk((HID, 16)),
                  _blk((HID, 16))],
        out_specs=[_rows(HID), _rows(HID), _rows(HID), _rows(16), _rows(16)],
        out_shape=[jax.ShapeDtypeStruct((NP, HID), jnp.float32)] * 3
        + [jax.ShapeDtypeStruct((NP, 16), jnp.float32)] * 2,
    )(x_p, w_in, b_in, w_t1, b_t1, w_t2, b_t2, wg0, asm0, adm0)


def _tc_mid_body(oa_ref, ob_ref, bg_ref, g_ref, b_ref, m_ref, v_ref,
                 wg_ref, asm_ref, adm_ref, xw_ref, als_ref, ald_ref):
    gs = oa_ref[...] + ob_ref[...] + bg_ref[...]
    xh = (gs - m_ref[...]) * lax.rsqrt(v_ref[...] + 1e-5) * g_ref[...] + b_ref[...]
    xh = jnp.where(xh > 0.0, xh, jnp.exp(xh) - 1.0)
    xw = _dot(xh, wg_ref[...])
    xw_ref[...] = xw
    als_ref[...] = _dot(xw, asm_ref[...])
    ald_ref[...] = _dot(xw, adm_ref[...])


def _tc_mid(oa, ob, bg, g, b, m, v, wg, asm, adm):
    w = wg.shape[1]
    return pl.pallas_call(
        _tc_mid_body,
        grid=(_G,),
        in_specs=[_rows(HID), _rows(HID)] + [_blk((1, HID))] * 5
        + [_blk((HID, w)), _blk((w, 16)), _blk((w, 16))],
        out_specs=[_rows(w), _rows(16), _rows(16)],
        out_shape=[jax.ShapeDtypeStruct((NP, w), jnp.float32),
                   jax.ShapeDtypeStruct((NP, 16), jnp.float32),
                   jax.ShapeDtypeStruct((NP, 16), jnp.float32)],
    )(oa, ob, bg, g, b, m, v, wg, asm, adm)


def _tc3_body(oa_ref, ob_ref, bg_ref, g_ref, b_ref, m_ref, v_ref, ht_ref,
              wc1a_ref, wc1b_ref, bc1_ref, wc2_ref, bc2_ref, out_ref):
    gs = oa_ref[...] + ob_ref[...] + bg_ref[...]
    hg = (gs - m_ref[...]) * lax.rsqrt(v_ref[...] + 1e-5) * g_ref[...] + b_ref[...]
    z = jnp.maximum(_dot(hg, wc1a_ref[...]) + _dot(ht_ref[...], wc1b_ref[...])
                    + bc1_ref[...], 0.0)
    out_ref[...] = _dot(z, wc2_ref[...]) + bc2_ref[...]


def _tc3(oa, ob, bg, g, b, m, v, ht, wc1a, wc1b, bc1, wc2, bc2):
    return pl.pallas_call(
        _tc3_body,
        grid=(_G,),
        in_specs=[_rows(HID), _rows(HID)] + [_blk((1, HID))] * 5
        + [_rows(HID), _blk((HID, HID)), _blk((HID, HID)), _blk((1, HID)),
           _blk((HID, 2)), _blk((1, 2))],
        out_specs=[_rows(2)],
        out_shape=[jax.ShapeDtypeStruct((NP, 2), jnp.float32)],
    )(oa, ob, bg, g, b, m, v, ht, wc1a, wc1b, bc1, wc2, bc2)


_MESH = plsc.VectorSubcoreMesh(core_axis_name="c", subcore_axis_name="s")

_CHA = 128
_NCHA = EPW // _CHA


@functools.partial(
    pl.kernel,
    out_type=(
        jax.ShapeDtypeStruct((EP, 16), jnp.float32),
        jax.ShapeDtypeStruct((NP, 16), jnp.float32),
        jax.ShapeDtypeStruct((NP, 16), jnp.float32),
    ),
    mesh=_MESH,
    compiler_params=pltpu.CompilerParams(use_tc_tiling_on_sc=False),
    scratch_types=[
        pltpu.VMEM((_NCHA, _CHA), jnp.int32),
        pltpu.VMEM((_NCHA, _CHA), jnp.int32),
        pltpu.VMEM((2, _CHA, 16), jnp.float32),
        pltpu.VMEM((2, _CHA, 16), jnp.float32),
        pltpu.VMEM((_CHA, 16), jnp.float32),
        pltpu.VMEM_SHARED((NP, 16), jnp.float32),
        pltpu.SemaphoreType.DMA,
        pltpu.SemaphoreType.DMA,
    ],
)
def _sc_phase_a(als_hbm, ald_hbm, src_hbm, dst_hbm, zero16_hbm,
                ee_hbm, den0_hbm, den1_hbm,
                idx_s, idx_d, gs_v, gd_v, ee_v, den_acc, sem0, sem1):
    c = lax.axis_index("c")
    s = lax.axis_index("s")
    wid = s * NC + c
    stripe = pl.ds(s * STRIPE, STRIPE)
    pltpu.sync_copy(zero16_hbm.at[stripe], den_acc.at[stripe])
    pltpu.sync_copy(src_hbm.at[wid], idx_s)
    pltpu.sync_copy(dst_hbm.at[wid], idx_d)
    plsc.subcore_barrier()
    sems = (sem0, sem1)

    def issue(k, b):
        pltpu.async_copy(als_hbm.at[idx_s.at[k]], gs_v.at[b], sems[b])
        pltpu.async_copy(ald_hbm.at[idx_d.at[k]], gd_v.at[b], sems[b])

    def wait(k, b):
        pltpu.make_async_copy(als_hbm.at[idx_s.at[k]], gs_v.at[b], sems[b]).wait()
        pltpu.make_async_copy(ald_hbm.at[idx_d.at[k]], gd_v.at[b], sems[b]).wait()

    def work(k, b):
        def row_body(r, _):
            e = gs_v[b, r, :] + gd_v[b, r, :]
            e = jnp.where(e >= 0.0, e, 0.2 * e)
            ee_v[r, :] = jnp.exp(e)
            return 0

        lax.fori_loop(0, _CHA, row_body, 0, unroll=4)
        base = wid * EPW + k * _CHA
        pltpu.sync_copy(ee_v, ee_hbm.at[pl.ds(base, _CHA)])
        pltpu.sync_copy(ee_v, den_acc.at[idx_d.at[k]], add=True)

    issue(0, 0)

    def pair_body(p, _):
        issue(2 * p + 1, 1)
        wait(2 * p, 0)
        work(2 * p, 0)

        @pl.when(2 * p + 2 < _NCHA)
        def _():
            issue(2 * p + 2, 0)

        wait(2 * p + 1, 1)
        work(2 * p + 1, 1)
        return 0

    lax.fori_loop(0, _NCHA // 2, pair_body, 0)
    plsc.subcore_barrier()

    @pl.when(c == 0)
    def _():
        pltpu.sync_copy(den_acc.at[stripe], den0_hbm.at[stripe])

    @pl.when(c == 1)
    def _():
        pltpu.sync_copy(den_acc.at[stripe], den1_hbm.at[stripe])


def _mk_phase_b(od):
    w = HEADS * od
    ch = 64 if od <= 16 else 16
    nch = EPW // ch
    scale = 1.0 if od <= 16 else 1.0 / HEADS

    @functools.partial(
        pl.kernel,
        out_type=(
            jax.ShapeDtypeStruct((NP, HID), jnp.float32),
            jax.ShapeDtypeStruct((NP, HID), jnp.float32),
        ),
        mesh=_MESH,
        compiler_params=pltpu.CompilerParams(use_tc_tiling_on_sc=False),
        scratch_types=[
            pltpu.VMEM((nch, ch), jnp.int32),
            pltpu.VMEM((nch, ch), jnp.int32),
            pltpu.VMEM((2, ch, w), jnp.float32),
            pltpu.VMEM((2, ch, 16), jnp.float32),
            pltpu.VMEM((2, ch, 16), jnp.float32),
            pltpu.VMEM((2, ch, 16), jnp.float32),
            pltpu.VMEM((ch, HID), jnp.float32),
            pltpu.VMEM_SHARED((NP, HID), jnp.float32),
            pltpu.SemaphoreType.DMA,
            pltpu.SemaphoreType.DMA,
        ],
    )
    def phase_b(xw_hbm, ee_hbm, den0_hbm, den1_hbm, src_hbm, dst_hbm,
                zero128_hbm, out0_hbm, out1_hbm,
                idx_s, idx_d, xw_v, ee_v, da_v, db_v, ct_v, out_acc,
                sem0, sem1):
        c = lax.axis_index("c")
        s = lax.axis_index("s")
        wid = s * NC + c
        stripe = pl.ds(s * STRIPE, STRIPE)
        pltpu.sync_copy(zero128_hbm.at[stripe], out_acc.at[stripe])
        pltpu.sync_copy(src_hbm.at[wid], idx_s)
        pltpu.sync_copy(dst_hbm.at[wid], idx_d)
        plsc.subcore_barrier()
        sems = (sem0, sem1)

        def issue(k, b):
            base = wid * EPW + k * ch
            pltpu.async_copy(xw_hbm.at[idx_s.at[k]], xw_v.at[b], sems[b])
            pltpu.async_copy(den0_hbm.at[idx_d.at[k]], da_v.at[b], sems[b])
            pltpu.async_copy(den1_hbm.at[idx_d.at[k]], db_v.at[b], sems[b])
            pltpu.async_copy(ee_hbm.at[pl.ds(base, ch)], ee_v.at[b], sems[b])

        def wait(k, b):
            base = wid * EPW + k * ch
            pltpu.make_async_copy(xw_hbm.at[idx_s.at[k]], xw_v.at[b],
                                  sems[b]).wait()
            pltpu.make_async_copy(den0_hbm.at[idx_d.at[k]], da_v.at[b],
                                  sems[b]).wait()
            pltpu.make_async_copy(den1_hbm.at[idx_d.at[k]], db_v.at[b],
                                  sems[b]).wait()
            pltpu.make_async_copy(ee_hbm.at[pl.ds(base, ch)], ee_v.at[b],
                                  sems[b]).wait()

        def work(k, b):
            def row_body(r, _):
                cf = (ee_v[b, r, :]
                      / (da_v[b, r, :] + db_v[b, r, :] + 1e-16) * scale)
                if od <= 16:
                    for j in range(HEADS):
                        ct_v[r, pl.ds(16 * j, 16)] = (
                            cf[j] * xw_v[b, r, pl.ds(16 * j, 16)])
                else:
                    for j in range(8):
                        p0 = cf[0] * xw_v[b, r, pl.ds(16 * j, 16)]
                        p1 = cf[1] * xw_v[b, r, pl.ds(HID + 16 * j, 16)]
                        p2 = cf[2] * xw_v[b, r, pl.ds(2 * HID + 16 * j, 16)]
                        p3 = cf[3] * xw_v[b, r, pl.ds(3 * HID + 16 * j, 16)]
                        p4 = cf[4] * xw_v[b, r, pl.ds(4 * HID + 16 * j, 16)]
                        p5 = cf[5] * xw_v[b, r, pl.ds(5 * HID + 16 * j, 16)]
                        p6 = cf[6] * xw_v[b, r, pl.ds(6 * HID + 16 * j, 16)]
                        p7 = cf[7] * xw_v[b, r, pl.ds(7 * HID + 16 * j, 16)]
                        ct_v[r, pl.ds(16 * j, 16)] = (
                            ((p0 + p1) + (p2 + p3))
                            + ((p4 + p5) + (p6 + p7)))
                return 0

            lax.fori_loop(0, ch, row_body, 0, unroll=2)
            pltpu.sync_copy(ct_v, out_acc.at[idx_d.at[k]], add=True)

        issue(0, 0)

        def pair_body(p, _):
            issue(2 * p + 1, 1)
            wait(2 * p, 0)
            work(2 * p, 0)

            @pl.when(2 * p + 2 < nch)
            def _():
                issue(2 * p + 2, 0)

            wait(2 * p + 1, 1)
            work(2 * p + 1, 1)
            return 0

        lax.fori_loop(0, nch // 2, pair_body, 0)
        plsc.subcore_barrier()

        @pl.when(c == 0)
        def _():
            pltpu.sync_copy(out_acc.at[stripe], out0_hbm.at[stripe])

        @pl.when(c == 1)
        def _():
            pltpu.sync_copy(out_acc.at[stripe], out1_hbm.at[stripe])

    return phase_b


_sc_phase_b16 = _mk_phase_b(16)
_sc_phase_b128 = _mk_phase_b(128)


def _att_mat(a):
    od = a.shape[1]
    mask = jnp.kron(jnp.eye(HEADS, dtype=jnp.float32), jnp.ones((od, 1), jnp.float32))
    vals = jnp.tile(a.T, (HEADS, 1))
    m = mask * vals
    return jnp.concatenate([m, m], axis=1)


def kernel(x, edge_index, w_in, b_in, w_t1, b_t1, w_t2, b_t2, wg0, as0, ad0,
           bg0, bn_g0, bn_b0, bn_m0, bn_v0, wg1, as1, ad1, bg1, bn_g1, bn_b1,
           bn_m1, bn_v1, wg2, as2, ad2, bg2, bn_g2, bn_b2, bn_m2, bn_v2,
           w_c1, b_c1, w_c2, b_c2):
    f32 = jnp.float32
    x_p = jnp.pad(x, ((0, NP - N), (0, 0)))
    loops = jnp.arange(N, dtype=jnp.int32)
    padi = jnp.full((EP - EF,), N, jnp.int32)
    src = jnp.concatenate([edge_index[0].astype(jnp.int32), loops, padi])
    dst = jnp.concatenate([edge_index[1].astype(jnp.int32), loops, padi])

    row = lambda v: v.reshape(1, -1).astype(f32)
    z16 = jnp.zeros((NP, 16), f32)
    z128 = jnp.zeros((NP, HID), f32)

    src_a = src.reshape(NW, _NCHA, _CHA)
    dst_a = dst.reshape(NW, _NCHA, _CHA)
    src_b16 = src.reshape(NW, EPW // 64, 64)
    dst_b16 = dst.reshape(NW, EPW // 64, 64)
    src_b128 = src.reshape(NW, EPW // 16, 16)
    dst_b128 = dst.reshape(NW, EPW // 16, 16)

    h, ht, xw0, als0, ald0 = _tc0(
        x_p, w_in, row(b_in), w_t1, row(b_t1), w_t2, row(b_t2), wg0,
        _att_mat(as0), _att_mat(ad0))

    ee0, dena0, denb0 = _sc_phase_a(als0, ald0, src_a, dst_a, z16)
    outa0, outb0 = _sc_phase_b16(xw0, ee0, dena0, denb0, src_b16, dst_b16, z128)

    xw1, als1, ald1 = _tc_mid(
        outa0, outb0, row(bg0), row(bn_g0), row(bn_b0), row(bn_m0),
        row(bn_v0), wg1, _att_mat(as1), _att_mat(ad1))

    ee1, dena1, denb1 = _sc_phase_a(als1, ald1, src_a, dst_a, z16)
    outa1, outb1 = _sc_phase_b16(xw1, ee1, dena1, denb1, src_b16, dst_b16, z128)

    xw2, als2, ald2 = _tc_mid(
        outa1, outb1, row(bg1), row(bn_g1), row(bn_b1), row(bn_m1),
        row(bn_v1), wg2, _att_mat(as2), _att_mat(ad2))

    ee2, dena2, denb2 = _sc_phase_a(als2, ald2, src_a, dst_a, z16)
    outa2, outb2 = _sc_phase_b128(xw2, ee2, dena2, denb2, src_b128, dst_b128, z128)

    out = _tc3(outa2, outb2, row(bg2), row(bn_g2), row(bn_b2), row(bn_m2),
               row(bn_v2), ht, w_c1[:HID], w_c1[HID:], row(b_c1), w_c2,
               row(b_c2))[0]
    return out[:N]


# speedup vs baseline: 41.8789x; 1.0857x over previous
import functools

import jax
import jax.numpy as jnp
from jax import lax
from jax.experimental import pallas as pl
from jax.experimental.pallas import tpu as pltpu
from jax.experimental.pallas import tpu_sc as plsc

N = 10000
D_IN = 128
HID = 128
HEADS = 8
E = 160000
EF = E + N
NP = 10240
EP = 172032
NC, NS = 2, 16
NW = NC * NS
EPW = EP // NW
STRIPE = NP // NS

_R = 1024
_G = NP // _R


def _blk(shape):
    return pl.BlockSpec(shape, lambda i: (0,) * len(shape))


def _rows(cols):
    return pl.BlockSpec((_R, cols), lambda i: (i, 0))


def _dot(a, b):
    return jnp.dot(a, b, preferred_element_type=jnp.float32)


def _tc0_body(x_ref, w_in_ref, b_in_ref, w_t1_ref, b_t1_ref, w_t2_ref,
              b_t2_ref, wg_ref, asm_ref, adm_ref,
              h_ref, ht_ref, xw_ref, als_ref, ald_ref):
    h = _dot(x_ref[...], w_in_ref[...]) + b_in_ref[...]
    h_ref[...] = h
    t = jnp.maximum(_dot(h, w_t1_ref[...]) + b_t1_ref[...], 0.0)
    ht_ref[...] = jnp.maximum(_dot(t, w_t2_ref[...]) + b_t2_ref[...], 0.0)
    xw = _dot(h, wg_ref[...])
    xw_ref[...] = xw
    als_ref[...] = _dot(xw, asm_ref[...])
    ald_ref[...] = _dot(xw, adm_ref[...])


def _tc0(x_p, w_in, b_in, w_t1, b_t1, w_t2, b_t2, wg0, asm0, adm0):
    return pl.pallas_call(
        _tc0_body,
        grid=(_G,),
        in_specs=[_rows(HID), _blk((HID, HID)), _blk((1, HID)),
                  _blk((HID, HID)), _blk((1, HID)), _blk((HID, HID)),
                  _blk((1, HID)), _blk((HID, HID)), _blk((HID, 16)),
                  _blk((HID, 16))],
        out_specs=[_rows(HID), _rows(HID), _rows(HID), _rows(16), _rows(16)],
        out_shape=[jax.ShapeDtypeStruct((NP, HID), jnp.float32)] * 3
        + [jax.ShapeDtypeStruct((NP, 16), jnp.float32)] * 2,
    )(x_p, w_in, b_in, w_t1, b_t1, w_t2, b_t2, wg0, asm0, adm0)


def _tc_mid_body(oa_ref, ob_ref, bg_ref, g_ref, b_ref, m_ref, v_ref,
                 wg_ref, asm_ref, adm_ref, xw_ref, als_ref, ald_ref):
    gs = oa_ref[...] + ob_ref[...] + bg_ref[...]
    xh = (gs - m_ref[...]) * lax.rsqrt(v_ref[...] + 1e-5) * g_ref[...] + b_ref[...]
    xh = jnp.where(xh > 0.0, xh, jnp.exp(xh) - 1.0)
    xw = _dot(xh, wg_ref[...])
    xw_ref[...] = xw.astype(xw_ref.dtype)
    als_ref[...] = _dot(xw, asm_ref[...])
    ald_ref[...] = _dot(xw, adm_ref[...])


def _tc_mid(oa, ob, bg, g, b, m, v, wg, asm, adm, xw_dtype=jnp.float32):
    w = wg.shape[1]
    return pl.pallas_call(
        _tc_mid_body,
        grid=(_G,),
        in_specs=[_rows(HID), _rows(HID)] + [_blk((1, HID))] * 5
        + [_blk((HID, w)), _blk((w, 16)), _blk((w, 16))],
        out_specs=[_rows(w), _rows(16), _rows(16)],
        out_shape=[jax.ShapeDtypeStruct((NP, w), xw_dtype),
                   jax.ShapeDtypeStruct((NP, 16), jnp.float32),
                   jax.ShapeDtypeStruct((NP, 16), jnp.float32)],
    )(oa, ob, bg, g, b, m, v, wg, asm, adm)


def _tc3_body(oa_ref, ob_ref, bg_ref, g_ref, b_ref, m_ref, v_ref, ht_ref,
              wc1a_ref, wc1b_ref, bc1_ref, wc2_ref, bc2_ref, out_ref):
    gs = oa_ref[...] + ob_ref[...] + bg_ref[...]
    hg = (gs - m_ref[...]) * lax.rsqrt(v_ref[...] + 1e-5) * g_ref[...] + b_ref[...]
    z = jnp.maximum(_dot(hg, wc1a_ref[...]) + _dot(ht_ref[...], wc1b_ref[...])
                    + bc1_ref[...], 0.0)
    out_ref[...] = _dot(z, wc2_ref[...]) + bc2_ref[...]


def _tc3(oa, ob, bg, g, b, m, v, ht, wc1a, wc1b, bc1, wc2, bc2):
    return pl.pallas_call(
        _tc3_body,
        grid=(_G,),
        in_specs=[_rows(HID), _rows(HID)] + [_blk((1, HID))] * 5
        + [_rows(HID), _blk((HID, HID)), _blk((HID, HID)), _blk((1, HID)),
           _blk((HID, 2)), _blk((1, 2))],
        out_specs=[_rows(2)],
        out_shape=[jax.ShapeDtypeStruct((NP, 2), jnp.float32)],
    )(oa, ob, bg, g, b, m, v, ht, wc1a, wc1b, bc1, wc2, bc2)


_MESH = plsc.VectorSubcoreMesh(core_axis_name="c", subcore_axis_name="s")

_CHA = 128
_NCHA = EPW // _CHA


@functools.partial(
    pl.kernel,
    out_type=(
        jax.ShapeDtypeStruct((EP, 16), jnp.float32),
        jax.ShapeDtypeStruct((NP, 16), jnp.float32),
        jax.ShapeDtypeStruct((NP, 16), jnp.float32),
    ),
    mesh=_MESH,
    compiler_params=pltpu.CompilerParams(use_tc_tiling_on_sc=False),
    scratch_types=[
        pltpu.VMEM((_NCHA, _CHA), jnp.int32),
        pltpu.VMEM((_NCHA, _CHA), jnp.int32),
        pltpu.VMEM((2, _CHA, 16), jnp.float32),
        pltpu.VMEM((2, _CHA, 16), jnp.float32),
        pltpu.VMEM((_CHA, 16), jnp.float32),
        pltpu.VMEM_SHARED((NP, 16), jnp.float32),
        pltpu.SemaphoreType.DMA,
        pltpu.SemaphoreType.DMA,
    ],
)
def _sc_phase_a(als_hbm, ald_hbm, src_hbm, dst_hbm, zero16_hbm,
                ee_hbm, den0_hbm, den1_hbm,
                idx_s, idx_d, gs_v, gd_v, ee_v, den_acc, sem0, sem1):
    c = lax.axis_index("c")
    s = lax.axis_index("s")
    wid = s * NC + c
    stripe = pl.ds(s * STRIPE, STRIPE)
    pltpu.sync_copy(zero16_hbm.at[stripe], den_acc.at[stripe])
    pltpu.sync_copy(src_hbm.at[wid], idx_s)
    pltpu.sync_copy(dst_hbm.at[wid], idx_d)
    plsc.subcore_barrier()
    sems = (sem0, sem1)

    def issue(k, b):
        pltpu.async_copy(als_hbm.at[idx_s.at[k]], gs_v.at[b], sems[b])
        pltpu.async_copy(ald_hbm.at[idx_d.at[k]], gd_v.at[b], sems[b])

    def wait(k, b):
        pltpu.make_async_copy(als_hbm.at[idx_s.at[k]], gs_v.at[b], sems[b]).wait()
        pltpu.make_async_copy(ald_hbm.at[idx_d.at[k]], gd_v.at[b], sems[b]).wait()

    def work(k, b):
        def row_body(r, _):
            e = gs_v[b, r, :] + gd_v[b, r, :]
            e = jnp.where(e >= 0.0, e, 0.2 * e)
            ee_v[r, :] = jnp.exp(e)
            return 0

        lax.fori_loop(0, _CHA, row_body, 0, unroll=4)
        base = wid * EPW + k * _CHA
        pltpu.sync_copy(ee_v, ee_hbm.at[pl.ds(base, _CHA)])
        pltpu.sync_copy(ee_v, den_acc.at[idx_d.at[k]], add=True)

    issue(0, 0)

    def pair_body(p, _):
        issue(2 * p + 1, 1)
        wait(2 * p, 0)
        work(2 * p, 0)

        @pl.when(2 * p + 2 < _NCHA)
        def _():
            issue(2 * p + 2, 0)

        wait(2 * p + 1, 1)
        work(2 * p + 1, 1)
        return 0

    lax.fori_loop(0, _NCHA // 2, pair_body, 0)
    plsc.subcore_barrier()

    @pl.when(c == 0)
    def _():
        pltpu.sync_copy(den_acc.at[stripe], den0_hbm.at[stripe])

    @pl.when(c == 1)
    def _():
        pltpu.sync_copy(den_acc.at[stripe], den1_hbm.at[stripe])


def _mk_phase_b(od):
    w = HEADS * od
    ch = 64 if od <= 16 else 16
    nch = EPW // ch
    scale = 1.0 if od <= 16 else 1.0 / HEADS
    xw_dtype = jnp.float32 if od <= 16 else jnp.bfloat16

    @functools.partial(
        pl.kernel,
        out_type=(
            jax.ShapeDtypeStruct((NP, HID), jnp.float32),
            jax.ShapeDtypeStruct((NP, HID), jnp.float32),
        ),
        mesh=_MESH,
        compiler_params=pltpu.CompilerParams(
            use_tc_tiling_on_sc=False,
            needs_layout_passes=(od <= 16)),
        scratch_types=[
            pltpu.VMEM((nch, ch), jnp.int32),
            pltpu.VMEM((nch, ch), jnp.int32),
            pltpu.VMEM((2, ch, w), xw_dtype),
            pltpu.VMEM((2, ch, 16), jnp.float32),
            pltpu.VMEM((2, ch, 16), jnp.float32),
            pltpu.VMEM((2, ch, 16), jnp.float32),
            pltpu.VMEM((ch, HID), jnp.float32),
            pltpu.VMEM_SHARED((NP, HID), jnp.float32),
            pltpu.SemaphoreType.DMA,
            pltpu.SemaphoreType.DMA,
        ],
    )
    def phase_b(xw_hbm, ee_hbm, den0_hbm, den1_hbm, src_hbm, dst_hbm,
                zero128_hbm, out0_hbm, out1_hbm,
                idx_s, idx_d, xw_v, ee_v, da_v, db_v, ct_v, out_acc,
                sem0, sem1):
        c = lax.axis_index("c")
        s = lax.axis_index("s")
        wid = s * NC + c
        stripe = pl.ds(s * STRIPE, STRIPE)
        pltpu.sync_copy(zero128_hbm.at[stripe], out_acc.at[stripe])
        pltpu.sync_copy(src_hbm.at[wid], idx_s)
        pltpu.sync_copy(dst_hbm.at[wid], idx_d)
        plsc.subcore_barrier()
        sems = (sem0, sem1)

        def issue(k, b):
            base = wid * EPW + k * ch
            pltpu.async_copy(xw_hbm.at[idx_s.at[k]], xw_v.at[b], sems[b])
            pltpu.async_copy(den0_hbm.at[idx_d.at[k]], da_v.at[b], sems[b])
            pltpu.async_copy(den1_hbm.at[idx_d.at[k]], db_v.at[b], sems[b])
            pltpu.async_copy(ee_hbm.at[pl.ds(base, ch)], ee_v.at[b], sems[b])

        def wait(k, b):
            base = wid * EPW + k * ch
            pltpu.make_async_copy(xw_hbm.at[idx_s.at[k]], xw_v.at[b],
                                  sems[b]).wait()
            pltpu.make_async_copy(den0_hbm.at[idx_d.at[k]], da_v.at[b],
                                  sems[b]).wait()
            pltpu.make_async_copy(den1_hbm.at[idx_d.at[k]], db_v.at[b],
                                  sems[b]).wait()
            pltpu.make_async_copy(ee_hbm.at[pl.ds(base, ch)], ee_v.at[b],
                                  sems[b]).wait()

        def work(k, b):
            def row_body(r, _):
                cf = (ee_v[b, r, :]
                      / (da_v[b, r, :] + db_v[b, r, :] + 1e-16) * scale)
                if od <= 16:
                    for j in range(HEADS):
                        ct_v[r, pl.ds(16 * j, 16)] = (
                            cf[j] * xw_v[b, r, pl.ds(16 * j, 16)])
                else:
                    for q in range(4):
                        p0, p1 = [], []
                        for h in range(HEADS):
                            v32 = xw_v[b, r, pl.ds(32 * (h * 4 + q), 32)]
                            lo, hi = plsc.unpack(
                                v32, format=plsc.PackFormat.INTERLEAVED)
                            p0.append(cf[h] * lo)
                            p1.append(cf[h] * hi)

                        def _tree(p):
                            return (((p[0] + p[1]) + (p[2] + p[3]))
                                    + ((p[4] + p[5]) + (p[6] + p[7])))

                        ct_v[r, pl.ds(32 * q, 16)] = _tree(p0)
                        ct_v[r, pl.ds(32 * q + 16, 16)] = _tree(p1)
                return 0

            lax.fori_loop(0, ch, row_body, 0, unroll=2)
            pltpu.sync_copy(ct_v, out_acc.at[idx_d.at[k]], add=True)

        issue(0, 0)

        def pair_body(p, _):
            issue(2 * p + 1, 1)
            wait(2 * p, 0)
            work(2 * p, 0)

            @pl.when(2 * p + 2 < nch)
            def _():
                issue(2 * p + 2, 0)

            wait(2 * p + 1, 1)
            work(2 * p + 1, 1)
            return 0

        lax.fori_loop(0, nch // 2, pair_body, 0)
        plsc.subcore_barrier()

        @pl.when(c == 0)
        def _():
            pltpu.sync_copy(out_acc.at[stripe], out0_hbm.at[stripe])

        @pl.when(c == 1)
        def _():
            pltpu.sync_copy(out_acc.at[stripe], out1_hbm.at[stripe])

    return phase_b


_sc_phase_b16 = _mk_phase_b(16)
_sc_phase_b128 = _mk_phase_b(128)


def _att_mat(a):
    od = a.shape[1]
    mask = jnp.kron(jnp.eye(HEADS, dtype=jnp.float32), jnp.ones((od, 1), jnp.float32))
    vals = jnp.tile(a.T, (HEADS, 1))
    m = mask * vals
    return jnp.concatenate([m, m], axis=1)


def kernel(x, edge_index, w_in, b_in, w_t1, b_t1, w_t2, b_t2, wg0, as0, ad0,
           bg0, bn_g0, bn_b0, bn_m0, bn_v0, wg1, as1, ad1, bg1, bn_g1, bn_b1,
           bn_m1, bn_v1, wg2, as2, ad2, bg2, bn_g2, bn_b2, bn_m2, bn_v2,
           w_c1, b_c1, w_c2, b_c2):
    f32 = jnp.float32
    x_p = jnp.pad(x, ((0, NP - N), (0, 0)))
    loops = jnp.arange(N, dtype=jnp.int32)
    padi = jnp.full((EP - EF,), N, jnp.int32)
    src = jnp.concatenate([edge_index[0].astype(jnp.int32), loops, padi])
    dst = jnp.concatenate([edge_index[1].astype(jnp.int32), loops, padi])

    row = lambda v: v.reshape(1, -1).astype(f32)
    z16 = jnp.zeros((NP, 16), f32)
    z128 = jnp.zeros((NP, HID), f32)

    src_a = src.reshape(NW, _NCHA, _CHA)
    dst_a = dst.reshape(NW, _NCHA, _CHA)
    src_b16 = src.reshape(NW, EPW // 64, 64)
    dst_b16 = dst.reshape(NW, EPW // 64, 64)
    src_b128 = src.reshape(NW, EPW // 16, 16)
    dst_b128 = dst.reshape(NW, EPW // 16, 16)

    h, ht, xw0, als0, ald0 = _tc0(
        x_p, w_in, row(b_in), w_t1, row(b_t1), w_t2, row(b_t2), wg0,
        _att_mat(as0), _att_mat(ad0))

    ee0, dena0, denb0 = _sc_phase_a(als0, ald0, src_a, dst_a, z16)
    outa0, outb0 = _sc_phase_b16(xw0, ee0, dena0, denb0, src_b16, dst_b16, z128)

    xw1, als1, ald1 = _tc_mid(
        outa0, outb0, row(bg0), row(bn_g0), row(bn_b0), row(bn_m0),
        row(bn_v0), wg1, _att_mat(as1), _att_mat(ad1))

    ee1, dena1, denb1 = _sc_phase_a(als1, ald1, src_a, dst_a, z16)
    outa1, outb1 = _sc_phase_b16(xw1, ee1, dena1, denb1, src_b16, dst_b16, z128)

    ci = jnp.arange(HEADS * HID, dtype=jnp.int32)
    blk, within = ci // 32, ci % 32
    permvec = blk * 32 + jnp.where(within % 2 == 0, within // 2,
                                   16 + within // 2)
    xw2, als2, ald2 = _tc_mid(
        outa1, outb1, row(bg1), row(bn_g1), row(bn_b1), row(bn_m1),
        row(bn_v1), wg2[:, permvec], _att_mat(as2)[permvec, :],
        _att_mat(ad2)[permvec, :], xw_dtype=jnp.bfloat16)

    ee2, dena2, denb2 = _sc_phase_a(als2, ald2, src_a, dst_a, z16)
    outa2, outb2 = _sc_phase_b128(xw2, ee2, dena2, denb2, src_b128, dst_b128, z128)

    out = _tc3(outa2, outb2, row(bg2), row(bn_g2), row(bn_b2), row(bn_m2),
               row(bn_v2), ht, w_c1[:HID], w_c1[HID:], row(b_c1), w_c2,
               row(b_c2))[0]
    return out[:N]


# speedup vs baseline: 53.6138x; 1.2802x over previous
import functools

import jax
import jax.numpy as jnp
from jax import lax
from jax.experimental import pallas as pl
from jax.experimental.pallas import tpu as pltpu
from jax.experimental.pallas import tpu_sc as plsc

N = 10000
D_IN = 128
HID = 128
HEADS = 8
E = 160000
EF = E + N
NP = 10240
EP = 172032
NC, NS = 2, 16
NW = NC * NS
EPW = EP // NW
STRIPE = NP // NS

_R = 1024
_G = NP // _R


def _blk(shape):
    return pl.BlockSpec(shape, lambda i: (0,) * len(shape))


def _rows(cols):
    return pl.BlockSpec((_R, cols), lambda i: (i, 0))


def _dot(a, b):
    return jnp.dot(a, b, preferred_element_type=jnp.float32)


def _tc0_body(x_ref, w_in_ref, b_in_ref, w_t1_ref, b_t1_ref, w_t2_ref,
              b_t2_ref, wg_ref, asm_ref, adm_ref,
              h_ref, ht_ref, xw_ref, als_ref, ald_ref):
    h = _dot(x_ref[...], w_in_ref[...]) + b_in_ref[...]
    h_ref[...] = h
    t = jnp.maximum(_dot(h, w_t1_ref[...]) + b_t1_ref[...], 0.0)
    ht_ref[...] = jnp.maximum(_dot(t, w_t2_ref[...]) + b_t2_ref[...], 0.0)
    xw = _dot(h, wg_ref[...])
    xw_ref[...] = xw.astype(xw_ref.dtype)
    als_ref[...] = _dot(xw, asm_ref[...])
    ald_ref[...] = _dot(xw, adm_ref[...])


def _tc0(x_p, w_in, b_in, w_t1, b_t1, w_t2, b_t2, wg0, asm0, adm0):
    return pl.pallas_call(
        _tc0_body,
        grid=(_G,),
        in_specs=[_rows(HID), _blk((HID, HID)), _blk((1, HID)),
                  _blk((HID, HID)), _blk((1, HID)), _blk((HID, HID)),
                  _blk((1, HID)), _blk((HID, HID)), _blk((HID, 16)),
                  _blk((HID, 16))],
        out_specs=[_rows(HID), _rows(HID), _rows(HID), _rows(16), _rows(16)],
        out_shape=[jax.ShapeDtypeStruct((NP, HID), jnp.float32)] * 2
        + [jax.ShapeDtypeStruct((NP, HID), jnp.bfloat16)]
        + [jax.ShapeDtypeStruct((NP, 16), jnp.float32)] * 2,
    )(x_p, w_in, b_in, w_t1, b_t1, w_t2, b_t2, wg0, asm0, adm0)


def _tc_mid_body(oa_ref, ob_ref, bg_ref, g_ref, b_ref, m_ref, v_ref,
                 wg_ref, asm_ref, adm_ref, xw_ref, als_ref, ald_ref):
    gs = oa_ref[...] + ob_ref[...] + bg_ref[...]
    xh = (gs - m_ref[...]) * lax.rsqrt(v_ref[...] + 1e-5) * g_ref[...] + b_ref[...]
    xh = jnp.where(xh > 0.0, xh, jnp.exp(xh) - 1.0)
    xw = _dot(xh, wg_ref[...])
    xw_ref[...] = xw.astype(xw_ref.dtype)
    als_ref[...] = _dot(xw, asm_ref[...])
    ald_ref[...] = _dot(xw, adm_ref[...])


def _tc_mid(oa, ob, bg, g, b, m, v, wg, asm, adm):
    w = wg.shape[1]
    return pl.pallas_call(
        _tc_mid_body,
        grid=(_G,),
        in_specs=[_rows(HID), _rows(HID)] + [_blk((1, HID))] * 5
        + [_blk((HID, w)), _blk((w, 16)), _blk((w, 16))],
        out_specs=[_rows(w), _rows(16), _rows(16)],
        out_shape=[jax.ShapeDtypeStruct((NP, w), jnp.bfloat16),
                   jax.ShapeDtypeStruct((NP, 16), jnp.float32),
                   jax.ShapeDtypeStruct((NP, 16), jnp.float32)],
    )(oa, ob, bg, g, b, m, v, wg, asm, adm)


def _tc_rden_body(a_ref, b_ref, r_ref):
    r_ref[...] = 1.0 / (a_ref[...] + b_ref[...] + 1e-16)


def _tc_rden(a, b):
    return pl.pallas_call(
        _tc_rden_body,
        grid=(_G,),
        in_specs=[_rows(16), _rows(16)],
        out_specs=[_rows(16)],
        out_shape=[jax.ShapeDtypeStruct((NP, 16), jnp.float32)],
    )(a, b)[0]


def _tc3_body(oa_ref, ob_ref, bg_ref, g_ref, b_ref, m_ref, v_ref, ht_ref,
              wc1a_ref, wc1b_ref, bc1_ref, wc2_ref, bc2_ref, out_ref):
    gs = oa_ref[...] + ob_ref[...] + bg_ref[...]
    hg = (gs - m_ref[...]) * lax.rsqrt(v_ref[...] + 1e-5) * g_ref[...] + b_ref[...]
    z = jnp.maximum(_dot(hg, wc1a_ref[...]) + _dot(ht_ref[...], wc1b_ref[...])
                    + bc1_ref[...], 0.0)
    out_ref[...] = _dot(z, wc2_ref[...]) + bc2_ref[...]


def _tc3(oa, ob, bg, g, b, m, v, ht, wc1a, wc1b, bc1, wc2, bc2):
    return pl.pallas_call(
        _tc3_body,
        grid=(_G,),
        in_specs=[_rows(HID), _rows(HID)] + [_blk((1, HID))] * 5
        + [_rows(HID), _blk((HID, HID)), _blk((HID, HID)), _blk((1, HID)),
           _blk((HID, 2)), _blk((1, 2))],
        out_specs=[_rows(2)],
        out_shape=[jax.ShapeDtypeStruct((NP, 2), jnp.float32)],
    )(oa, ob, bg, g, b, m, v, ht, wc1a, wc1b, bc1, wc2, bc2)


_MESH = plsc.VectorSubcoreMesh(core_axis_name="c", subcore_axis_name="s")

_CHA = 128
_NCHA = EPW // _CHA


@functools.partial(
    pl.kernel,
    out_type=(
        jax.ShapeDtypeStruct((EP, 16), jnp.float32),
        jax.ShapeDtypeStruct((NP, 16), jnp.float32),
        jax.ShapeDtypeStruct((NP, 16), jnp.float32),
    ),
    mesh=_MESH,
    compiler_params=pltpu.CompilerParams(use_tc_tiling_on_sc=False),
    scratch_types=[
        pltpu.VMEM((_NCHA, _CHA), jnp.int32),
        pltpu.VMEM((_NCHA, _CHA), jnp.int32),
        pltpu.VMEM((2, _CHA, 16), jnp.float32),
        pltpu.VMEM((2, _CHA, 16), jnp.float32),
        pltpu.VMEM((_CHA, 16), jnp.float32),
        pltpu.VMEM_SHARED((NP, 16), jnp.float32),
        pltpu.SemaphoreType.DMA,
        pltpu.SemaphoreType.DMA,
    ],
)
def _sc_phase_a(als_hbm, ald_hbm, src_hbm, dst_hbm, zero16_hbm,
                ee_hbm, den0_hbm, den1_hbm,
                idx_s, idx_d, gs_v, gd_v, ee_v, den_acc, sem0, sem1):
    c = lax.axis_index("c")
    s = lax.axis_index("s")
    wid = s * NC + c
    stripe = pl.ds(s * STRIPE, STRIPE)
    pltpu.sync_copy(zero16_hbm.at[stripe], den_acc.at[stripe])
    pltpu.sync_copy(src_hbm.at[wid], idx_s)
    pltpu.sync_copy(dst_hbm.at[wid], idx_d)
    plsc.subcore_barrier()
    sems = (sem0, sem1)

    def issue(k, b):
        pltpu.async_copy(als_hbm.at[idx_s.at[k]], gs_v.at[b], sems[b])
        pltpu.async_copy(ald_hbm.at[idx_d.at[k]], gd_v.at[b], sems[b])

    def wait(k, b):
        pltpu.make_async_copy(als_hbm.at[idx_s.at[k]], gs_v.at[b], sems[b]).wait()
        pltpu.make_async_copy(ald_hbm.at[idx_d.at[k]], gd_v.at[b], sems[b]).wait()

    def work(k, b):
        def row_body(r, _):
            e = gs_v[b, r, :] + gd_v[b, r, :]
            e = jnp.where(e >= 0.0, e, 0.2 * e)
            ee_v[r, :] = jnp.exp(e)
            return 0

        lax.fori_loop(0, _CHA, row_body, 0, unroll=4)
        base = wid * EPW + k * _CHA
        pltpu.sync_copy(ee_v, ee_hbm.at[pl.ds(base, _CHA)])
        pltpu.sync_copy(ee_v, den_acc.at[idx_d.at[k]], add=True)

    issue(0, 0)

    def pair_body(p, _):
        issue(2 * p + 1, 1)
        wait(2 * p, 0)
        work(2 * p, 0)

        @pl.when(2 * p + 2 < _NCHA)
        def _():
            issue(2 * p + 2, 0)

        wait(2 * p + 1, 1)
        work(2 * p + 1, 1)
        return 0

    lax.fori_loop(0, _NCHA // 2, pair_body, 0)
    plsc.subcore_barrier()

    @pl.when(c == 0)
    def _():
        pltpu.sync_copy(den_acc.at[stripe], den0_hbm.at[stripe])

    @pl.when(c == 1)
    def _():
        pltpu.sync_copy(den_acc.at[stripe], den1_hbm.at[stripe])


def _mk_phase_b(od):
    w = HEADS * od
    ch = 64 if od <= 16 else 16
    nch = EPW // ch
    scale = 1.0 if od <= 16 else 1.0 / HEADS

    @functools.partial(
        pl.kernel,
        out_type=(
            jax.ShapeDtypeStruct((NP, HID), jnp.float32),
            jax.ShapeDtypeStruct((NP, HID), jnp.float32),
        ),
        mesh=_MESH,
        compiler_params=pltpu.CompilerParams(
            use_tc_tiling_on_sc=False, needs_layout_passes=False),
        scratch_types=[
            pltpu.VMEM((nch, ch), jnp.int32),
            pltpu.VMEM((nch, ch), jnp.int32),
            pltpu.VMEM((2, ch, w), jnp.bfloat16),
            pltpu.VMEM((2, ch, 16), jnp.float32),
            pltpu.VMEM((2, ch, 16), jnp.float32),
            pltpu.VMEM((2, ch, HID), jnp.float32),
            pltpu.VMEM_SHARED((NP, HID), jnp.float32),
            pltpu.SemaphoreType.DMA,
            pltpu.SemaphoreType.DMA,
            pltpu.SemaphoreType.DMA,
            pltpu.SemaphoreType.DMA,
        ],
    )
    def phase_b(xw_hbm, ee_hbm, rden_hbm, src_hbm, dst_hbm,
                zero128_hbm, out0_hbm, out1_hbm,
                idx_s, idx_d, xw_v, ee_v, rd_v, ct_v, out_acc,
                sem0, sem1, ssem0, ssem1):
        c = lax.axis_index("c")
        s = lax.axis_index("s")
        wid = s * NC + c
        stripe = pl.ds(s * STRIPE, STRIPE)
        pltpu.sync_copy(zero128_hbm.at[stripe], out_acc.at[stripe])
        pltpu.sync_copy(src_hbm.at[wid], idx_s)
        pltpu.sync_copy(dst_hbm.at[wid], idx_d)
        plsc.subcore_barrier()
        sems = (sem0, sem1)
        ssems = (ssem0, ssem1)

        def issue(k, b):
            base = wid * EPW + k * ch
            pltpu.async_copy(xw_hbm.at[idx_s.at[k]], xw_v.at[b], sems[b])
            pltpu.async_copy(rden_hbm.at[idx_d.at[k]], rd_v.at[b], sems[b])
            pltpu.async_copy(ee_hbm.at[pl.ds(base, ch)], ee_v.at[b], sems[b])

        def wait(k, b):
            base = wid * EPW + k * ch
            pltpu.make_async_copy(xw_hbm.at[idx_s.at[k]], xw_v.at[b],
                                  sems[b]).wait()
            pltpu.make_async_copy(rden_hbm.at[idx_d.at[k]], rd_v.at[b],
                                  sems[b]).wait()
            pltpu.make_async_copy(ee_hbm.at[pl.ds(base, ch)], ee_v.at[b],
                                  sems[b]).wait()

        def work(k, b):
            def row_body(r, _):
                cf = ee_v[b, r, :] * rd_v[b, r, :] * scale
                if od <= 16:
                    for q in range(4):
                        v32 = xw_v[b, r, pl.ds(32 * q, 32)]
                        lo, hi = plsc.unpack(
                            v32, format=plsc.PackFormat.INTERLEAVED)
                        ct_v[b, r, pl.ds(32 * q, 16)] = cf[2 * q] * lo
                        ct_v[b, r, pl.ds(32 * q + 16, 16)] = (
                            cf[2 * q + 1] * hi)
                else:
                    for q in range(4):
                        p0, p1 = [], []
                        for h in range(HEADS):
                            v32 = xw_v[b, r, pl.ds(32 * (h * 4 + q), 32)]
                            lo, hi = plsc.unpack(
                                v32, format=plsc.PackFormat.INTERLEAVED)
                            p0.append(cf[h] * lo)
                            p1.append(cf[h] * hi)

                        def _tree(p):
                            return (((p[0] + p[1]) + (p[2] + p[3]))
                                    + ((p[4] + p[5]) + (p[6] + p[7])))

                        ct_v[b, r, pl.ds(32 * q, 16)] = _tree(p0)
                        ct_v[b, r, pl.ds(32 * q + 16, 16)] = _tree(p1)
                return 0

            lax.fori_loop(0, ch, row_body, 0, unroll=2)
            pltpu.async_copy(ct_v.at[b], out_acc.at[idx_d.at[k]], ssems[b],
                             add=True)

        def wait_scatter(k, b):
            pltpu.make_async_copy(ct_v.at[b], out_acc.at[idx_d.at[k]],
                                  ssems[b]).wait()

        issue(0, 0)

        def pair_body(p, _):
            issue(2 * p + 1, 1)
            wait(2 * p, 0)

            @pl.when(p >= 1)
            def _():
                wait_scatter(2 * p - 2, 0)

            work(2 * p, 0)

            @pl.when(2 * p + 2 < nch)
            def _():
                issue(2 * p + 2, 0)

            wait(2 * p + 1, 1)

            @pl.when(p >= 1)
            def _():
                wait_scatter(2 * p - 1, 1)

            work(2 * p + 1, 1)
            return 0

        lax.fori_loop(0, nch // 2, pair_body, 0)
        wait_scatter(nch - 2, 0)
        wait_scatter(nch - 1, 1)
        plsc.subcore_barrier()

        @pl.when(c == 0)
        def _():
            pltpu.sync_copy(out_acc.at[stripe], out0_hbm.at[stripe])

        @pl.when(c == 1)
        def _():
            pltpu.sync_copy(out_acc.at[stripe], out1_hbm.at[stripe])

    return phase_b


_sc_phase_b16 = _mk_phase_b(16)
_sc_phase_b128 = _mk_phase_b(128)


def _ilv_perm(n):
    ci = jnp.arange(n, dtype=jnp.int32)
    blk, w_ = ci // 32, ci % 32
    return blk * 32 + jnp.where(w_ % 2 == 0, w_ // 2, 16 + w_ // 2)


def _att_mat(a):
    od = a.shape[1]
    mask = jnp.kron(jnp.eye(HEADS, dtype=jnp.float32), jnp.ones((od, 1), jnp.float32))
    vals = jnp.tile(a.T, (HEADS, 1))
    m = mask * vals
    return jnp.concatenate([m, m], axis=1)


def kernel(x, edge_index, w_in, b_in, w_t1, b_t1, w_t2, b_t2, wg0, as0, ad0,
           bg0, bn_g0, bn_b0, bn_m0, bn_v0, wg1, as1, ad1, bg1, bn_g1, bn_b1,
           bn_m1, bn_v1, wg2, as2, ad2, bg2, bn_g2, bn_b2, bn_m2, bn_v2,
           w_c1, b_c1, w_c2, b_c2):
    f32 = jnp.float32
    x_p = jnp.pad(x, ((0, NP - N), (0, 0)))
    loops = jnp.arange(N, dtype=jnp.int32)
    padi = jnp.full((EP - EF,), N, jnp.int32)
    src = jnp.concatenate([edge_index[0].astype(jnp.int32), loops, padi])
    dst = jnp.concatenate([edge_index[1].astype(jnp.int32), loops, padi])

    row = lambda v: v.reshape(1, -1).astype(f32)
    z16 = jnp.zeros((NP, 16), f32)
    z128 = jnp.zeros((NP, HID), f32)

    src_a = src.reshape(NW, _NCHA, _CHA)
    dst_a = dst.reshape(NW, _NCHA, _CHA)
    src_b16 = src.reshape(NW, EPW // 64, 64)
    dst_b16 = dst.reshape(NW, EPW // 64, 64)
    src_b128 = src.reshape(NW, EPW // 16, 16)
    dst_b128 = dst.reshape(NW, EPW // 16, 16)

    p128 = _ilv_perm(HID)
    p1024 = _ilv_perm(HEADS * HID)

    h, ht, xw0, als0, ald0 = _tc0(
        x_p, w_in, row(b_in), w_t1, row(b_t1), w_t2, row(b_t2),
        wg0[:, p128], _att_mat(as0)[p128, :], _att_mat(ad0)[p128, :])

    ee0, dena0, denb0 = _sc_phase_a(als0, ald0, src_a, dst_a, z16)
    outa0, outb0 = _sc_phase_b16(xw0, ee0, _tc_rden(dena0, denb0),
                                 src_b16, dst_b16, z128)

    xw1, als1, ald1 = _tc_mid(
        outa0, outb0, row(bg0), row(bn_g0), row(bn_b0), row(bn_m0),
        row(bn_v0), wg1[:, p128], _att_mat(as1)[p128, :],
        _att_mat(ad1)[p128, :])

    ee1, dena1, denb1 = _sc_phase_a(als1, ald1, src_a, dst_a, z16)
    outa1, outb1 = _sc_phase_b16(xw1, ee1, _tc_rden(dena1, denb1),
                                 src_b16, dst_b16, z128)

    xw2, als2, ald2 = _tc_mid(
        outa1, outb1, row(bg1), row(bn_g1), row(bn_b1), row(bn_m1),
        row(bn_v1), wg2[:, p1024], _att_mat(as2)[p1024, :],
        _att_mat(ad2)[p1024, :])

    ee2, dena2, denb2 = _sc_phase_a(als2, ald2, src_a, dst_a, z16)
    outa2, outb2 = _sc_phase_b128(xw2, ee2, _tc_rden(dena2, denb2),
                                  src_b128, dst_b128, z128)

    out = _tc3(outa2, outb2, row(bg2), row(bn_g2), row(bn_b2), row(bn_m2),
               row(bn_v2), ht, w_c1[:HID], w_c1[HID:], row(b_c1), w_c2,
               row(b_c2))[0]
    return out[:N]


# speedup vs baseline: 58.2338x; 1.0862x over previous
import functools

import jax
import jax.numpy as jnp
from jax import lax
from jax.experimental import pallas as pl
from jax.experimental.pallas import tpu as pltpu
from jax.experimental.pallas import tpu_sc as plsc

N = 10000
D_IN = 128
HID = 128
HEADS = 8
E = 160000
EF = E + N
NP = 10240
EP = 172032
NC, NS = 2, 16
NW = NC * NS
EPW = EP // NW
STRIPE = NP // NS

_R = 1024
_G = NP // _R


def _blk(shape):
    return pl.BlockSpec(shape, lambda i: (0,) * len(shape))


def _rows(cols):
    return pl.BlockSpec((_R, cols), lambda i: (i, 0))


def _dot(a, b):
    return jnp.dot(a, b, preferred_element_type=jnp.float32)


def _tc0_body(x_ref, w_in_ref, b_in_ref, w_t1_ref, b_t1_ref, w_t2_ref,
              b_t2_ref, wg_ref, asm_ref, adm_ref,
              h_ref, ht_ref, xw_ref, als_ref, ald_ref):
    h = _dot(x_ref[...], w_in_ref[...]) + b_in_ref[...]
    h_ref[...] = h
    t = jnp.maximum(_dot(h, w_t1_ref[...]) + b_t1_ref[...], 0.0)
    ht_ref[...] = jnp.maximum(_dot(t, w_t2_ref[...]) + b_t2_ref[...], 0.0)
    xw = _dot(h, wg_ref[...])
    xw_ref[...] = xw.astype(xw_ref.dtype)
    als_ref[...] = _dot(xw, asm_ref[...])
    ald_ref[...] = _dot(xw, adm_ref[...])


def _tc0(x_p, w_in, b_in, w_t1, b_t1, w_t2, b_t2, wg0, asm0, adm0):
    return pl.pallas_call(
        _tc0_body,
        grid=(_G,),
        in_specs=[_rows(HID), _blk((HID, HID)), _blk((1, HID)),
                  _blk((HID, HID)), _blk((1, HID)), _blk((HID, HID)),
                  _blk((1, HID)), _blk((HID, HID)), _blk((HID, 16)),
                  _blk((HID, 16))],
        out_specs=[_rows(HID), _rows(HID), _rows(HID), _rows(16), _rows(16)],
        out_shape=[jax.ShapeDtypeStruct((NP, HID), jnp.float32)] * 2
        + [jax.ShapeDtypeStruct((NP, HID), jnp.bfloat16)]
        + [jax.ShapeDtypeStruct((NP, 16), jnp.float32)] * 2,
    )(x_p, w_in, b_in, w_t1, b_t1, w_t2, b_t2, wg0, asm0, adm0)


def _tc_mid_body(oa_ref, ob_ref, bg_ref, g_ref, b_ref, m_ref, v_ref,
                 wg_ref, asm_ref, adm_ref, xw_ref, als_ref, ald_ref):
    gs = oa_ref[...] + ob_ref[...] + bg_ref[...]
    xh = (gs - m_ref[...]) * lax.rsqrt(v_ref[...] + 1e-5) * g_ref[...] + b_ref[...]
    xh = jnp.where(xh > 0.0, xh, jnp.exp(xh) - 1.0)
    xw = _dot(xh, wg_ref[...])
    xw_ref[...] = xw.astype(xw_ref.dtype)
    als_ref[...] = _dot(xw, asm_ref[...])
    ald_ref[...] = _dot(xw, adm_ref[...])


def _tc_mid(oa, ob, bg, g, b, m, v, wg, asm, adm):
    w = wg.shape[1]
    return pl.pallas_call(
        _tc_mid_body,
        grid=(_G,),
        in_specs=[_rows(HID), _rows(HID)] + [_blk((1, HID))] * 5
        + [_blk((HID, w)), _blk((w, 16)), _blk((w, 16))],
        out_specs=[_rows(w), _rows(16), _rows(16)],
        out_shape=[jax.ShapeDtypeStruct((NP, w), jnp.bfloat16),
                   jax.ShapeDtypeStruct((NP, 16), jnp.float32),
                   jax.ShapeDtypeStruct((NP, 16), jnp.float32)],
    )(oa, ob, bg, g, b, m, v, wg, asm, adm)


def _tc_rden_body(a_ref, b_ref, r_ref):
    r_ref[...] = 1.0 / (a_ref[...] + b_ref[...] + 1e-16)


def _tc_rden(a, b):
    return pl.pallas_call(
        _tc_rden_body,
        grid=(_G,),
        in_specs=[_rows(16), _rows(16)],
        out_specs=[_rows(16)],
        out_shape=[jax.ShapeDtypeStruct((NP, 16), jnp.float32)],
    )(a, b)[0]


def _tc3_body(oa_ref, ob_ref, bg_ref, g_ref, b_ref, m_ref, v_ref, ht_ref,
              wc1a_ref, wc1b_ref, bc1_ref, wc2_ref, bc2_ref, out_ref):
    gs = oa_ref[...] + ob_ref[...] + bg_ref[...]
    hg = (gs - m_ref[...]) * lax.rsqrt(v_ref[...] + 1e-5) * g_ref[...] + b_ref[...]
    z = jnp.maximum(_dot(hg, wc1a_ref[...]) + _dot(ht_ref[...], wc1b_ref[...])
                    + bc1_ref[...], 0.0)
    out_ref[...] = _dot(z, wc2_ref[...]) + bc2_ref[...]


def _tc3(oa, ob, bg, g, b, m, v, ht, wc1a, wc1b, bc1, wc2, bc2):
    return pl.pallas_call(
        _tc3_body,
        grid=(_G,),
        in_specs=[_rows(HID), _rows(HID)] + [_blk((1, HID))] * 5
        + [_rows(HID), _blk((HID, HID)), _blk((HID, HID)), _blk((1, HID)),
           _blk((HID, 2)), _blk((1, 2))],
        out_specs=[_rows(2)],
        out_shape=[jax.ShapeDtypeStruct((NP, 2), jnp.float32)],
    )(oa, ob, bg, g, b, m, v, ht, wc1a, wc1b, bc1, wc2, bc2)


_MESH = plsc.VectorSubcoreMesh(core_axis_name="c", subcore_axis_name="s")

_CHA = 128
_NCHA = EPW // _CHA


@functools.partial(
    pl.kernel,
    out_type=(
        jax.ShapeDtypeStruct((EP, 16), jnp.float32),
        jax.ShapeDtypeStruct((NP, 16), jnp.float32),
        jax.ShapeDtypeStruct((NP, 16), jnp.float32),
    ),
    mesh=_MESH,
    compiler_params=pltpu.CompilerParams(use_tc_tiling_on_sc=False),
    scratch_types=[
        pltpu.VMEM((_NCHA, _CHA), jnp.int32),
        pltpu.VMEM((_NCHA, _CHA), jnp.int32),
        pltpu.VMEM((2, _CHA, 16), jnp.float32),
        pltpu.VMEM((2, _CHA, 16), jnp.float32),
        pltpu.VMEM((2, _CHA, 16), jnp.float32),
        pltpu.VMEM_SHARED((NP, 16), jnp.float32),
        pltpu.SemaphoreType.DMA,
        pltpu.SemaphoreType.DMA,
        pltpu.SemaphoreType.DMA,
        pltpu.SemaphoreType.DMA,
        pltpu.SemaphoreType.DMA,
        pltpu.SemaphoreType.DMA,
    ],
)
def _sc_phase_a(als_hbm, ald_hbm, src_hbm, dst_hbm, zero16_hbm,
                ee_hbm, den0_hbm, den1_hbm,
                idx_s, idx_d, gs_v, gd_v, ee_v, den_acc,
                sem0, sem1, ssem0, ssem1, dsem0, dsem1):
    c = lax.axis_index("c")
    s = lax.axis_index("s")
    wid = s * NC + c
    stripe = pl.ds(s * STRIPE, STRIPE)
    pltpu.sync_copy(zero16_hbm.at[stripe], den_acc.at[stripe])
    pltpu.sync_copy(src_hbm.at[wid], idx_s)
    pltpu.sync_copy(dst_hbm.at[wid], idx_d)
    plsc.subcore_barrier()
    sems = (sem0, sem1)
    ssems = (ssem0, ssem1)
    dsems = (dsem0, dsem1)

    def issue(k, b):
        pltpu.async_copy(als_hbm.at[idx_s.at[k]], gs_v.at[b], sems[b])
        pltpu.async_copy(ald_hbm.at[idx_d.at[k]], gd_v.at[b], sems[b])

    def wait(k, b):
        pltpu.make_async_copy(als_hbm.at[idx_s.at[k]], gs_v.at[b], sems[b]).wait()
        pltpu.make_async_copy(ald_hbm.at[idx_d.at[k]], gd_v.at[b], sems[b]).wait()

    def work(k, b):
        def row_body(r, _):
            e = gs_v[b, r, :] + gd_v[b, r, :]
            e = jnp.where(e >= 0.0, e, 0.2 * e)
            ee_v[b, r, :] = jnp.exp(e)
            return 0

        lax.fori_loop(0, _CHA, row_body, 0, unroll=4)
        base = wid * EPW + k * _CHA
        pltpu.async_copy(ee_v.at[b], ee_hbm.at[pl.ds(base, _CHA)], ssems[b])
        pltpu.async_copy(ee_v.at[b], den_acc.at[idx_d.at[k]], dsems[b],
                         add=True)

    def wait_store(k, b):
        base = wid * EPW + k * _CHA
        pltpu.make_async_copy(ee_v.at[b], ee_hbm.at[pl.ds(base, _CHA)],
                              ssems[b]).wait()
        pltpu.make_async_copy(ee_v.at[b], den_acc.at[idx_d.at[k]],
                              dsems[b]).wait()

    issue(0, 0)

    def pair_body(p, _):
        issue(2 * p + 1, 1)
        wait(2 * p, 0)

        @pl.when(p >= 1)
        def _():
            wait_store(2 * p - 2, 0)

        work(2 * p, 0)

        @pl.when(2 * p + 2 < _NCHA)
        def _():
            issue(2 * p + 2, 0)

        wait(2 * p + 1, 1)

        @pl.when(p >= 1)
        def _():
            wait_store(2 * p - 1, 1)

        work(2 * p + 1, 1)
        return 0

    lax.fori_loop(0, _NCHA // 2, pair_body, 0)
    wait_store(_NCHA - 2, 0)
    wait_store(_NCHA - 1, 1)
    plsc.subcore_barrier()

    @pl.when(c == 0)
    def _():
        pltpu.sync_copy(den_acc.at[stripe], den0_hbm.at[stripe])

    @pl.when(c == 1)
    def _():
        pltpu.sync_copy(den_acc.at[stripe], den1_hbm.at[stripe])


def _mk_phase_b(od):
    w = HEADS * od
    ch = 64 if od <= 16 else 16
    nch = EPW // ch
    scale = 1.0 if od <= 16 else 1.0 / HEADS

    @functools.partial(
        pl.kernel,
        out_type=(
            jax.ShapeDtypeStruct((NP, HID), jnp.float32),
            jax.ShapeDtypeStruct((NP, HID), jnp.float32),
        ),
        mesh=_MESH,
        compiler_params=pltpu.CompilerParams(
            use_tc_tiling_on_sc=False, needs_layout_passes=False),
        scratch_types=[
            pltpu.VMEM((nch, ch), jnp.int32),
            pltpu.VMEM((nch, ch), jnp.int32),
            pltpu.VMEM((2, ch, w), jnp.bfloat16),
            pltpu.VMEM((2, ch, 16), jnp.float32),
            pltpu.VMEM((2, ch, 16), jnp.float32),
            pltpu.VMEM((2, ch, HID), jnp.float32),
            pltpu.VMEM_SHARED((NP, HID), jnp.float32),
            pltpu.SemaphoreType.DMA,
            pltpu.SemaphoreType.DMA,
            pltpu.SemaphoreType.DMA,
            pltpu.SemaphoreType.DMA,
        ],
    )
    def phase_b(xw_hbm, ee_hbm, rden_hbm, src_hbm, dst_hbm,
                zero128_hbm, out0_hbm, out1_hbm,
                idx_s, idx_d, xw_v, ee_v, rd_v, ct_v, out_acc,
                sem0, sem1, ssem0, ssem1):
        c = lax.axis_index("c")
        s = lax.axis_index("s")
        wid = s * NC + c
        stripe = pl.ds(s * STRIPE, STRIPE)
        pltpu.sync_copy(zero128_hbm.at[stripe], out_acc.at[stripe])
        pltpu.sync_copy(src_hbm.at[wid], idx_s)
        pltpu.sync_copy(dst_hbm.at[wid], idx_d)
        plsc.subcore_barrier()
        sems = (sem0, sem1)
        ssems = (ssem0, ssem1)

        def issue(k, b):
            base = wid * EPW + k * ch
            pltpu.async_copy(xw_hbm.at[idx_s.at[k]], xw_v.at[b], sems[b])
            pltpu.async_copy(rden_hbm.at[idx_d.at[k]], rd_v.at[b], sems[b])
            pltpu.async_copy(ee_hbm.at[pl.ds(base, ch)], ee_v.at[b], sems[b])

        def wait(k, b):
            base = wid * EPW + k * ch
            pltpu.make_async_copy(xw_hbm.at[idx_s.at[k]], xw_v.at[b],
                                  sems[b]).wait()
            pltpu.make_async_copy(rden_hbm.at[idx_d.at[k]], rd_v.at[b],
                                  sems[b]).wait()
            pltpu.make_async_copy(ee_hbm.at[pl.ds(base, ch)], ee_v.at[b],
                                  sems[b]).wait()

        def work(k, b):
            def row_body(r, _):
                cf = ee_v[b, r, :] * rd_v[b, r, :] * scale
                if od <= 16:
                    for q in range(4):
                        v32 = xw_v[b, r, pl.ds(32 * q, 32)]
                        lo, hi = plsc.unpack(
                            v32, format=plsc.PackFormat.INTERLEAVED)
                        ct_v[b, r, pl.ds(32 * q, 16)] = cf[2 * q] * lo
                        ct_v[b, r, pl.ds(32 * q + 16, 16)] = (
                            cf[2 * q + 1] * hi)
                else:
                    for q in range(4):
                        p0, p1 = [], []
                        for h in range(HEADS):
                            v32 = xw_v[b, r, pl.ds(32 * (h * 4 + q), 32)]
                            lo, hi = plsc.unpack(
                                v32, format=plsc.PackFormat.INTERLEAVED)
                            p0.append(cf[h] * lo)
                            p1.append(cf[h] * hi)

                        def _tree(p):
                            return (((p[0] + p[1]) + (p[2] + p[3]))
                                    + ((p[4] + p[5]) + (p[6] + p[7])))

                        ct_v[b, r, pl.ds(32 * q, 16)] = _tree(p0)
                        ct_v[b, r, pl.ds(32 * q + 16, 16)] = _tree(p1)
                return 0

            lax.fori_loop(0, ch, row_body, 0, unroll=2)
            pltpu.async_copy(ct_v.at[b], out_acc.at[idx_d.at[k]], ssems[b],
                             add=True)

        def wait_scatter(k, b):
            pltpu.make_async_copy(ct_v.at[b], out_acc.at[idx_d.at[k]],
                                  ssems[b]).wait()

        issue(0, 0)

        def pair_body(p, _):
            issue(2 * p + 1, 1)
            wait(2 * p, 0)

            @pl.when(p >= 1)
            def _():
                wait_scatter(2 * p - 2, 0)

            work(2 * p, 0)

            @pl.when(2 * p + 2 < nch)
            def _():
                issue(2 * p + 2, 0)

            wait(2 * p + 1, 1)

            @pl.when(p >= 1)
            def _():
                wait_scatter(2 * p - 1, 1)

            work(2 * p + 1, 1)
            return 0

        lax.fori_loop(0, nch // 2, pair_body, 0)
        wait_scatter(nch - 2, 0)
        wait_scatter(nch - 1, 1)
        plsc.subcore_barrier()

        @pl.when(c == 0)
        def _():
            pltpu.sync_copy(out_acc.at[stripe], out0_hbm.at[stripe])

        @pl.when(c == 1)
        def _():
            pltpu.sync_copy(out_acc.at[stripe], out1_hbm.at[stripe])

    return phase_b


_sc_phase_b16 = _mk_phase_b(16)
_sc_phase_b128 = _mk_phase_b(128)


def _ilv_perm(n):
    ci = jnp.arange(n, dtype=jnp.int32)
    blk, w_ = ci // 32, ci % 32
    return blk * 32 + jnp.where(w_ % 2 == 0, w_ // 2, 16 + w_ // 2)


def _att_mat(a):
    od = a.shape[1]
    mask = jnp.kron(jnp.eye(HEADS, dtype=jnp.float32), jnp.ones((od, 1), jnp.float32))
    vals = jnp.tile(a.T, (HEADS, 1))
    m = mask * vals
    return jnp.concatenate([m, m], axis=1)


def kernel(x, edge_index, w_in, b_in, w_t1, b_t1, w_t2, b_t2, wg0, as0, ad0,
           bg0, bn_g0, bn_b0, bn_m0, bn_v0, wg1, as1, ad1, bg1, bn_g1, bn_b1,
           bn_m1, bn_v1, wg2, as2, ad2, bg2, bn_g2, bn_b2, bn_m2, bn_v2,
           w_c1, b_c1, w_c2, b_c2):
    f32 = jnp.float32
    x_p = jnp.pad(x, ((0, NP - N), (0, 0)))
    loops = jnp.arange(N, dtype=jnp.int32)
    padi = N + jnp.arange(EP - EF, dtype=jnp.int32) % (NP - N)
    src = jnp.concatenate([edge_index[0].astype(jnp.int32), loops, padi])
    dst = jnp.concatenate([edge_index[1].astype(jnp.int32), loops, padi])

    row = lambda v: v.reshape(1, -1).astype(f32)
    z16 = jnp.zeros((NP, 16), f32)
    z128 = jnp.zeros((NP, HID), f32)

    src_a = src.reshape(NW, _NCHA, _CHA)
    dst_a = dst.reshape(NW, _NCHA, _CHA)
    src_b16 = src.reshape(NW, EPW // 64, 64)
    dst_b16 = dst.reshape(NW, EPW // 64, 64)
    src_b128 = src.reshape(NW, EPW // 16, 16)
    dst_b128 = dst.reshape(NW, EPW // 16, 16)

    p128 = _ilv_perm(HID)
    p1024 = _ilv_perm(HEADS * HID)

    h, ht, xw0, als0, ald0 = _tc0(
        x_p, w_in, row(b_in), w_t1, row(b_t1), w_t2, row(b_t2),
        wg0[:, p128], _att_mat(as0)[p128, :], _att_mat(ad0)[p128, :])

    ee0, dena0, denb0 = _sc_phase_a(als0, ald0, src_a, dst_a, z16)
    outa0, outb0 = _sc_phase_b16(xw0, ee0, _tc_rden(dena0, denb0),
                                 src_b16, dst_b16, z128)

    xw1, als1, ald1 = _tc_mid(
        outa0, outb0, row(bg0), row(bn_g0), row(bn_b0), row(bn_m0),
        row(bn_v0), wg1[:, p128], _att_mat(as1)[p128, :],
        _att_mat(ad1)[p128, :])

    ee1, dena1, denb1 = _sc_phase_a(als1, ald1, src_a, dst_a, z16)
    outa1, outb1 = _sc_phase_b16(xw1, ee1, _tc_rden(dena1, denb1),
                                 src_b16, dst_b16, z128)

    xw2, als2, ald2 = _tc_mid(
        outa1, outb1, row(bg1), row(bn_g1), row(bn_b1), row(bn_m1),
        row(bn_v1), wg2[:, p1024], _att_mat(as2)[p1024, :],
        _att_mat(ad2)[p1024, :])

    ee2, dena2, denb2 = _sc_phase_a(als2, ald2, src_a, dst_a, z16)
    outa2, outb2 = _sc_phase_b128(xw2, ee2, _tc_rden(dena2, denb2),
                                  src_b128, dst_b128, z128)

    out = _tc3(outa2, outb2, row(bg2), row(bn_g2), row(bn_b2), row(bn_m2),
               row(bn_v2), ht, w_c1[:HID], w_c1[HID:], row(b_c1), w_c2,
               row(b_c2))[0]
    return out[:N]
